# Initial kernel scaffold; baseline (speedup 1.0000x reference)
import jax
import jax.numpy as jnp
from jax.experimental import pallas as pl


def kernel(x, edge_index, edge_attr, batch, W0, b0, We1, be1, We2, be2, Wroot, bconv, W_ih, W_hh, b_ih, b_hh, Wl_ih, Wl_hh, bl_ih, bl_hh, W1, b1):
    raise NotImplementedError("write your pallas kernel here")


# speedup vs baseline: 1.1676x; 1.1676x over previous
import functools

import jax
import jax.numpy as jnp
from jax import lax
from jax.experimental import pallas as pl
from jax.experimental.pallas import tpu as pltpu

N = 10000
E = 160000
NUM_FEAT = 128
DIM = 32
B = 312
NEG_SLOPE = 0.01

NP = 10240
EP = 163840
EB = 512
CH = 1280
NC = NP // CH


def _leaky(v):
    return jnp.where(v >= 0, v, NEG_SLOPE * v)


def _encode_body(x_ref, w0t_ref, b0_ref, o_ref):
    v = _leaky(jnp.dot(x_ref[...], w0t_ref[...],
                       preferred_element_type=jnp.float32) + b0_ref[...])
    row = lax.broadcasted_iota(jnp.int32, (NP, 1), 0)
    o_ref[...] = jnp.where(row < N, v, 0.0)


def _encode(x_p, W0T, b0):
    return pl.pallas_call(
        _encode_body,
        out_shape=jax.ShapeDtypeStruct((NP, DIM), jnp.float32),
    )(x_p, W0T, b0.reshape(1, DIM))


def _emlp_body(ea_ref, w1t_ref, b1_ref, o_ref):
    o_ref[...] = _leaky(jnp.dot(ea_ref[...], w1t_ref[...],
                                preferred_element_type=jnp.float32) + b1_ref[...])


def _edge_mlp(ea_p, We1T, be1):
    blk = 8192
    return pl.pallas_call(
        _emlp_body,
        grid=(EP // blk,),
        in_specs=[
            pl.BlockSpec((blk, 4), lambda i: (i, 0)),
            pl.BlockSpec((4, 128), lambda i: (0, 0)),
            pl.BlockSpec((1, 128), lambda i: (0, 0)),
        ],
        out_specs=pl.BlockSpec((blk, 128), lambda i: (i, 0)),
        out_shape=jax.ShapeDtypeStruct((EP, 128), jnp.float32),
    )(ea_p, We1T, be1.reshape(1, 128))


def _msg_body(h_ref, xj_ref, w2t_ref, b2_ref, s_ref, o_ref):
    w = jnp.dot(h_ref[...], w2t_ref[...],
                preferred_element_type=jnp.float32) + b2_ref[...]
    xt = jnp.tile(xj_ref[...], (1, DIM))
    o_ref[...] = jnp.dot(w * xt, s_ref[...], preferred_element_type=jnp.float32)


def _messages(h_e, xj, We2pT, be2p, S):
    return pl.pallas_call(
        _msg_body,
        grid=(EP // EB,),
        in_specs=[
            pl.BlockSpec((EB, 128), lambda i: (i, 0)),
            pl.BlockSpec((EB, DIM), lambda i: (i, 0)),
            pl.BlockSpec((128, DIM * DIM), lambda i: (0, 0)),
            pl.BlockSpec((1, DIM * DIM), lambda i: (0, 0)),
            pl.BlockSpec((DIM * DIM, DIM), lambda i: (0, 0)),
        ],
        out_specs=pl.BlockSpec((EB, DIM), lambda i: (i, 0)),
        out_shape=jax.ShapeDtypeStruct((EP, DIM), jnp.float32),
    )(h_e, xj, We2pT, be2p, S)


def _update_body(p_ref, c_ref, x_ref, h_ref, wroot_ref, bconv_ref,
                 wih_ref, bih_ref, whh_ref, bhh_ref, o_ref):
    s = p_ref[0] + p_ref[1]
    cnt = c_ref[0][:, :1] + c_ref[1][:, :1]
    mean = s / jnp.maximum(cnt, 1.0)
    conv = (jnp.dot(x_ref[...], wroot_ref[...],
                    preferred_element_type=jnp.float32) + mean + bconv_ref[...])
    m = _leaky(conv)
    gi = jnp.dot(m, wih_ref[...], preferred_element_type=jnp.float32) + bih_ref[...]
    gh = jnp.dot(h_ref[...], whh_ref[...], preferred_element_type=jnp.float32) + bhh_ref[...]
    r = jax.nn.sigmoid(gi[:, :DIM] + gh[:, :DIM])
    z = jax.nn.sigmoid(gi[:, DIM:2 * DIM] + gh[:, DIM:2 * DIM])
    n = jnp.tanh(gi[:, 2 * DIM:] + r * gh[:, 2 * DIM:])
    hn = (1.0 - z) * n + z * h_ref[...]
    row = lax.broadcasted_iota(jnp.int32, (NP, 1), 0)
    o_ref[...] = jnp.where(row < N, hn, 0.0)


def _update(parts, cparts, x_cur, h_prev, WrootT, bconv, W_ihT, b_ih, W_hhT, b_hh):
    return pl.pallas_call(
        _update_body,
        out_shape=jax.ShapeDtypeStruct((NP, DIM), jnp.float32),
    )(parts, cparts, x_cur, h_prev, WrootT, bconv.reshape(1, DIM),
      W_ihT, b_ih.reshape(1, 3 * DIM), W_hhT, b_hh.reshape(1, 3 * DIM))


def _s2s_body(x_ref, b_ref, wlih_ref, blih_ref, wlhh_ref, blhh_ref,
              w1t_ref, b1_ref, o_ref, e_ref):
    q_star = jnp.zeros((B, 2 * DIM), jnp.float32)
    hx = jnp.zeros((B, DIM), jnp.float32)
    cx = jnp.zeros((B, DIM), jnp.float32)
    for _ in range(3):
        gates = (jnp.dot(q_star, wlih_ref[...], preferred_element_type=jnp.float32)
                 + blih_ref[...]
                 + jnp.dot(hx, wlhh_ref[...], preferred_element_type=jnp.float32)
                 + blhh_ref[...])
        i_g = jax.nn.sigmoid(gates[:, :DIM])
        f_g = jax.nn.sigmoid(gates[:, DIM:2 * DIM])
        g_g = jnp.tanh(gates[:, 2 * DIM:3 * DIM])
        o_g = jax.nn.sigmoid(gates[:, 3 * DIM:])
        cx = f_g * cx + i_g * g_g
        hx = o_g * jnp.tanh(cx)
        q = hx

        gid = lax.broadcasted_iota(jnp.int32, (CH, B), 1)
        emax = jnp.full((B,), -1e30, jnp.float32)
        for c in range(NC):
            x_c = x_ref[c * CH:(c + 1) * CH, :]
            b_c = b_ref[c * CH:(c + 1) * CH, :]
            mk = (b_c == gid)
            mf = mk.astype(jnp.float32)
            qg = jnp.dot(mf, q, preferred_element_type=jnp.float32)
            e_c = jnp.sum(x_c * qg, axis=1)
            em = jnp.where(mk, e_c[:, None], -1e30)
            emax = jnp.maximum(emax, jnp.max(em, axis=0))
            e_ref[c, :] = e_c
        denom = jnp.zeros((B,), jnp.float32)
        racc = jnp.zeros((B, DIM), jnp.float32)
        for c in range(NC):
            x_c = x_ref[c * CH:(c + 1) * CH, :]
            b_c = b_ref[c * CH:(c + 1) * CH, :]
            mf = (b_c == gid).astype(jnp.float32)
            me = jnp.sum(mf * emax[None, :], axis=1)
            ex = jnp.exp(e_ref[c, :] - me)
            denom = denom + jnp.dot(ex, mf, preferred_element_type=jnp.float32)
            racc = racc + lax.dot_general(mf, ex[:, None] * x_c,
                                          (((0,), (0,)), ((), ())),
                                          preferred_element_type=jnp.float32)
        r = racc / jnp.maximum(denom, 1e-30)[:, None]
        q_star = jnp.concatenate([q, r], axis=1)
    o_ref[...] = jnp.dot(q_star, w1t_ref[...],
                         preferred_element_type=jnp.float32) + b1_ref[...]


def _set2set(x_nodes, batch_p, Wl_ihT, bl_ih, Wl_hhT, bl_hh, W1T, b1):
    return pl.pallas_call(
        _s2s_body,
        out_shape=jax.ShapeDtypeStruct((B, 1), jnp.float32),
        scratch_shapes=[pltpu.VMEM((NC, CH), jnp.float32)],
    )(x_nodes, batch_p, Wl_ihT, bl_ih.reshape(1, 4 * DIM),
      Wl_hhT, bl_hh.reshape(1, 4 * DIM), W1T, b1.reshape(1, 1))


def kernel(x, edge_index, edge_attr, batch, W0, b0, We1, be1, We2, be2,
           Wroot, bconv, W_ih, W_hh, b_ih, b_hh, Wl_ih, Wl_hh, bl_ih, bl_hh,
           W1, b1):
    f32 = jnp.float32
    x_p = jnp.zeros((NP, NUM_FEAT), f32).at[:N].set(x)
    src_p = jnp.full((EP,), N, jnp.int32).at[:E].set(edge_index[0])
    dst_p = jnp.zeros((EP,), jnp.int32).at[:E].set(edge_index[1])
    ea_p = jnp.zeros((EP, 4), f32).at[:E].set(edge_attr)
    batch_p = jnp.full((NP, 1), B, jnp.int32).at[:N, 0].set(batch)

    We2p = We2.reshape(DIM, DIM, 128).transpose(1, 0, 2).reshape(DIM * DIM, 128)
    be2p = be2.reshape(DIM, DIM).T.reshape(1, DIM * DIM)
    S = jnp.repeat(jnp.eye(DIM, dtype=f32), DIM, axis=0)

    ones_msk = (jnp.arange(EP) < E).astype(f32)[:, None] * jnp.ones((1, DIM), f32)

    out = _encode(x_p, W0.T, b0)
    h_e = _edge_mlp(ea_p, We1.T, be1)

    csum = jax.ops.segment_sum(ones_msk, dst_p, num_segments=NP)
    cparts = jnp.stack([csum, jnp.zeros_like(csum)])

    h = out
    for _ in range(2):
        xj = jnp.take(out, src_p, axis=0)
        msg = _messages(h_e, xj, We2p.T, be2p, S)
        ssum = jax.ops.segment_sum(msg, dst_p, num_segments=NP)
        parts = jnp.stack([ssum, jnp.zeros_like(ssum)])
        h = _update(parts, cparts, out, h, Wroot, bconv,
                    W_ih.T, b_ih, W_hh.T, b_hh)
        out = h

    return _set2set(out, batch_p, Wl_ih.T, bl_ih, Wl_hh.T, bl_hh, W1.T, b1)


# speedup vs baseline: 2.5769x; 2.2070x over previous
import functools

import jax
import jax.numpy as jnp
from jax import lax
from jax.experimental import pallas as pl
from jax.experimental.pallas import tpu as pltpu
from jax.experimental.pallas import tpu_sc as plsc

N = 10000
E = 160000
NUM_FEAT = 128
DIM = 32
B = 312
NEG_SLOPE = 0.01

NP = 10240
EP = 163840
EB = 512
CH = 1280
NC = NP // CH


def _leaky(v):
    return jnp.where(v >= 0, v, NEG_SLOPE * v)


def _encode_body(x_ref, w0t_ref, b0_ref, o_ref):
    v = _leaky(jnp.dot(x_ref[...], w0t_ref[...],
                       preferred_element_type=jnp.float32) + b0_ref[...])
    row = lax.broadcasted_iota(jnp.int32, (NP, 1), 0)
    o_ref[...] = jnp.where(row < N, v, 0.0)


def _encode(x_p, W0T, b0):
    return pl.pallas_call(
        _encode_body,
        out_shape=jax.ShapeDtypeStruct((NP, DIM), jnp.float32),
    )(x_p, W0T, b0.reshape(1, DIM))


def _emlp_body(ea_ref, w1t_ref, b1_ref, o_ref):
    o_ref[...] = _leaky(jnp.dot(ea_ref[...], w1t_ref[...],
                                preferred_element_type=jnp.float32) + b1_ref[...])


def _edge_mlp(ea_p, We1T, be1):
    blk = 8192
    return pl.pallas_call(
        _emlp_body,
        grid=(EP // blk,),
        in_specs=[
            pl.BlockSpec((blk, 4), lambda i: (i, 0)),
            pl.BlockSpec((4, 128), lambda i: (0, 0)),
            pl.BlockSpec((1, 128), lambda i: (0, 0)),
        ],
        out_specs=pl.BlockSpec((blk, 128), lambda i: (i, 0)),
        out_shape=jax.ShapeDtypeStruct((EP, 128), jnp.float32),
    )(ea_p, We1T, be1.reshape(1, 128))


def _msg_body(h_ref, xj_ref, w2t_ref, b2_ref, s_ref, o_ref):
    w = jnp.dot(h_ref[...], w2t_ref[...],
                preferred_element_type=jnp.float32) + b2_ref[...]
    xt = jnp.tile(xj_ref[...], (1, DIM))
    o_ref[...] = jnp.dot(w * xt, s_ref[...], preferred_element_type=jnp.float32)


def _messages(h_e, xj, We2pT, be2p, S):
    return pl.pallas_call(
        _msg_body,
        grid=(EP // EB,),
        in_specs=[
            pl.BlockSpec((EB, 128), lambda i: (i, 0)),
            pl.BlockSpec((EB, DIM), lambda i: (i, 0)),
            pl.BlockSpec((128, DIM * DIM), lambda i: (0, 0)),
            pl.BlockSpec((1, DIM * DIM), lambda i: (0, 0)),
            pl.BlockSpec((DIM * DIM, DIM), lambda i: (0, 0)),
        ],
        out_specs=pl.BlockSpec((EB, DIM), lambda i: (i, 0)),
        out_shape=jax.ShapeDtypeStruct((EP, DIM), jnp.float32),
    )(h_e, xj, We2pT, be2p, S)


def _update_body(p_ref, c_ref, x_ref, h_ref, wroot_ref, bconv_ref,
                 wih_ref, bih_ref, whh_ref, bhh_ref, o_ref):
    s = p_ref[0] + p_ref[1]
    cnt = c_ref[0][:, :1] + c_ref[1][:, :1]
    mean = s / jnp.maximum(cnt, 1.0)
    conv = (jnp.dot(x_ref[...], wroot_ref[...],
                    preferred_element_type=jnp.float32) + mean + bconv_ref[...])
    m = _leaky(conv)
    gi = jnp.dot(m, wih_ref[...], preferred_element_type=jnp.float32) + bih_ref[...]
    gh = jnp.dot(h_ref[...], whh_ref[...], preferred_element_type=jnp.float32) + bhh_ref[...]
    r = jax.nn.sigmoid(gi[:, :DIM] + gh[:, :DIM])
    z = jax.nn.sigmoid(gi[:, DIM:2 * DIM] + gh[:, DIM:2 * DIM])
    n = jnp.tanh(gi[:, 2 * DIM:] + r * gh[:, 2 * DIM:])
    hn = (1.0 - z) * n + z * h_ref[...]
    row = lax.broadcasted_iota(jnp.int32, (NP, 1), 0)
    o_ref[...] = jnp.where(row < N, hn, 0.0)


def _update(parts, cparts, x_cur, h_prev, WrootT, bconv, W_ihT, b_ih, W_hhT, b_hh):
    return pl.pallas_call(
        _update_body,
        out_shape=jax.ShapeDtypeStruct((NP, DIM), jnp.float32),
    )(parts, cparts, x_cur, h_prev, WrootT, bconv.reshape(1, DIM),
      W_ihT, b_ih.reshape(1, 3 * DIM), W_hhT, b_hh.reshape(1, 3 * DIM))


def _s2s_body(x_ref, b_ref, wlih_ref, blih_ref, wlhh_ref, blhh_ref,
              w1t_ref, b1_ref, o_ref, e_ref):
    q_star = jnp.zeros((B, 2 * DIM), jnp.float32)
    hx = jnp.zeros((B, DIM), jnp.float32)
    cx = jnp.zeros((B, DIM), jnp.float32)
    for _ in range(3):
        gates = (jnp.dot(q_star, wlih_ref[...], preferred_element_type=jnp.float32)
                 + blih_ref[...]
                 + jnp.dot(hx, wlhh_ref[...], preferred_element_type=jnp.float32)
                 + blhh_ref[...])
        i_g = jax.nn.sigmoid(gates[:, :DIM])
        f_g = jax.nn.sigmoid(gates[:, DIM:2 * DIM])
        g_g = jnp.tanh(gates[:, 2 * DIM:3 * DIM])
        o_g = jax.nn.sigmoid(gates[:, 3 * DIM:])
        cx = f_g * cx + i_g * g_g
        hx = o_g * jnp.tanh(cx)
        q = hx

        gid = lax.broadcasted_iota(jnp.int32, (CH, B), 1)
        emax = jnp.full((B,), -1e30, jnp.float32)
        for c in range(NC):
            x_c = x_ref[c * CH:(c + 1) * CH, :]
            b_c = b_ref[c * CH:(c + 1) * CH, :]
            mk = (b_c == gid)
            mf = mk.astype(jnp.float32)
            qg = jnp.dot(mf, q, preferred_element_type=jnp.float32)
            e_c = jnp.sum(x_c * qg, axis=1)
            em = jnp.where(mk, e_c[:, None], -1e30)
            emax = jnp.maximum(emax, jnp.max(em, axis=0))
            e_ref[c, :] = e_c
        denom = jnp.zeros((B,), jnp.float32)
        racc = jnp.zeros((B, DIM), jnp.float32)
        for c in range(NC):
            x_c = x_ref[c * CH:(c + 1) * CH, :]
            b_c = b_ref[c * CH:(c + 1) * CH, :]
            mf = (b_c == gid).astype(jnp.float32)
            me = jnp.sum(mf * emax[None, :], axis=1)
            ex = jnp.exp(e_ref[c, :] - me)
            denom = denom + jnp.dot(ex, mf, preferred_element_type=jnp.float32)
            racc = racc + lax.dot_general(mf, ex[:, None] * x_c,
                                          (((0,), (0,)), ((), ())),
                                          preferred_element_type=jnp.float32)
        r = racc / jnp.maximum(denom, 1e-30)[:, None]
        q_star = jnp.concatenate([q, r], axis=1)
    o_ref[...] = jnp.dot(q_star, w1t_ref[...],
                         preferred_element_type=jnp.float32) + b1_ref[...]


def _set2set(x_nodes, batch_p, Wl_ihT, bl_ih, Wl_hhT, bl_hh, W1T, b1):
    return pl.pallas_call(
        _s2s_body,
        out_shape=jax.ShapeDtypeStruct((B, 1), jnp.float32),
        scratch_shapes=[pltpu.VMEM((NC, CH), jnp.float32)],
    )(x_nodes, batch_p, Wl_ihT, bl_ih.reshape(1, 4 * DIM),
      Wl_hhT, bl_hh.reshape(1, 4 * DIM), W1T, b1.reshape(1, 1))


NW = 32
NJ = EP // NW // 128
RPT = NP // 16

_SC_MESH = plsc.VectorSubcoreMesh(core_axis_name="c", subcore_axis_name="s")
_SC_PARAMS = pltpu.CompilerParams(use_tc_tiling_on_sc=False)


def _sc_gather_body(x_hbm, idx_hbm, out_hbm, idx_v, rows_v, sem):
    wid = lax.axis_index("c") * 16 + lax.axis_index("s")
    pltpu.sync_copy(idx_hbm.at[wid], idx_v)

    @pl.loop(0, NJ)
    def _(j):
        pltpu.async_copy(x_hbm.at[idx_v.at[j]], rows_v, sem).wait()
        pltpu.sync_copy(rows_v, out_hbm.at[wid, j])


def _sc_gather(x_cur, idx_r):
    fn = pl.kernel(
        _sc_gather_body,
        out_type=jax.ShapeDtypeStruct((NW, NJ, 128, DIM), jnp.float32),
        mesh=_SC_MESH,
        compiler_params=_SC_PARAMS,
        scratch_types=[
            pltpu.VMEM((NJ, 128), jnp.int32),
            pltpu.VMEM((128, DIM), jnp.float32),
            pltpu.SemaphoreType.DMA,
        ],
    )
    return fn(x_cur, idx_r).reshape(EP, DIM)


def _sc_scatter_body(msg_hbm, idx_hbm, zero_hbm, out_hbm, idx_v, row_v, acc, sem):
    cid = lax.axis_index("c")
    sid = lax.axis_index("s")
    wid = cid * 16 + sid
    pltpu.sync_copy(zero_hbm.at[pl.ds(sid * RPT, RPT)],
                    acc.at[pl.ds(sid * RPT, RPT)])
    plsc.subcore_barrier()
    pltpu.sync_copy(idx_hbm.at[wid], idx_v)

    @pl.loop(0, NJ)
    def _(j):
        pltpu.sync_copy(msg_hbm.at[wid, j], row_v)
        pltpu.sync_copy(row_v, acc.at[idx_v.at[j]], add=True)

    plsc.subcore_barrier()
    pltpu.sync_copy(acc.at[pl.ds(sid * RPT, RPT)],
                    out_hbm.at[cid, pl.ds(sid * RPT, RPT)])


def _sc_scatter(msg, idx_r, zero):
    fn = pl.kernel(
        _sc_scatter_body,
        out_type=jax.ShapeDtypeStruct((2, NP, DIM), jnp.float32),
        mesh=_SC_MESH,
        compiler_params=_SC_PARAMS,
        scratch_types=[
            pltpu.VMEM((NJ, 128), jnp.int32),
            pltpu.VMEM((128, DIM), jnp.float32),
            pltpu.VMEM_SHARED((NP, DIM), jnp.float32),
            pltpu.SemaphoreType.DMA,
        ],
    )
    return fn(msg.reshape(NW, NJ, 128, DIM), idx_r, zero)


def kernel(x, edge_index, edge_attr, batch, W0, b0, We1, be1, We2, be2,
           Wroot, bconv, W_ih, W_hh, b_ih, b_hh, Wl_ih, Wl_hh, bl_ih, bl_hh,
           W1, b1):
    f32 = jnp.float32
    x_p = jnp.zeros((NP, NUM_FEAT), f32).at[:N].set(x)
    src_p = jnp.full((EP,), N, jnp.int32).at[:E].set(edge_index[0])
    dst_p = jnp.zeros((EP,), jnp.int32).at[:E].set(edge_index[1])
    ea_p = jnp.zeros((EP, 4), f32).at[:E].set(edge_attr)
    batch_p = jnp.full((NP, 1), B, jnp.int32).at[:N, 0].set(batch)

    We2p = We2.reshape(DIM, DIM, 128).transpose(1, 0, 2).reshape(DIM * DIM, 128)
    be2p = be2.reshape(DIM, DIM).T.reshape(1, DIM * DIM)
    S = jnp.repeat(jnp.eye(DIM, dtype=f32), DIM, axis=0)

    ones_msk = (jnp.arange(EP) < E).astype(f32)[:, None] * jnp.ones((1, DIM), f32)

    src_r = src_p.reshape(NW, NJ, 128)
    dst_r = dst_p.reshape(NW, NJ, 128)
    zero = jnp.zeros((NP, DIM), f32)

    out = _encode(x_p, W0.T, b0)
    h_e = _edge_mlp(ea_p, We1.T, be1)

    cparts = _sc_scatter(ones_msk, dst_r, zero)

    h = out
    for _ in range(2):
        xj = _sc_gather(out, src_r)
        msg = _messages(h_e, xj, We2p.T, be2p, S)
        parts = _sc_scatter(msg, dst_r, zero)
        h = _update(parts, cparts, out, h, Wroot, bconv,
                    W_ih.T, b_ih, W_hh.T, b_hh)
        out = h

    return _set2set(out, batch_p, Wl_ih.T, bl_ih, Wl_hh.T, bl_hh, W1.T, b1)


# speedup vs baseline: 2.6433x; 1.0258x over previous
import functools

import jax
import jax.numpy as jnp
from jax import lax
from jax.experimental import pallas as pl
from jax.experimental.pallas import tpu as pltpu
from jax.experimental.pallas import tpu_sc as plsc

N = 10000
E = 160000
NUM_FEAT = 128
DIM = 32
B = 312
NEG_SLOPE = 0.01

NP = 10240
EP = 163840
EB = 512
CH = 1280
NC = NP // CH


def _leaky(v):
    return jnp.where(v >= 0, v, NEG_SLOPE * v)


def _encode_body(x_ref, w0t_ref, b0_ref, o_ref):
    v = _leaky(jnp.dot(x_ref[...], w0t_ref[...],
                       preferred_element_type=jnp.float32) + b0_ref[...])
    row = lax.broadcasted_iota(jnp.int32, (NP, 1), 0)
    o_ref[...] = jnp.where(row < N, v, 0.0)


def _encode(x_p, W0T, b0):
    return pl.pallas_call(
        _encode_body,
        out_shape=jax.ShapeDtypeStruct((NP, DIM), jnp.float32),
    )(x_p, W0T, b0.reshape(1, DIM))


def _emlp_body(ea_ref, w1t_ref, b1_ref, o_ref):
    o_ref[...] = _leaky(jnp.dot(ea_ref[...], w1t_ref[...],
                                preferred_element_type=jnp.float32)
                        + b1_ref[...]).astype(jnp.bfloat16)


def _edge_mlp(ea_p, We1T, be1):
    blk = 8192
    return pl.pallas_call(
        _emlp_body,
        grid=(EP // blk,),
        in_specs=[
            pl.BlockSpec((blk, 4), lambda i: (i, 0)),
            pl.BlockSpec((4, 128), lambda i: (0, 0)),
            pl.BlockSpec((1, 128), lambda i: (0, 0)),
        ],
        out_specs=pl.BlockSpec((blk, 128), lambda i: (i, 0)),
        out_shape=jax.ShapeDtypeStruct((EP, 128), jnp.bfloat16),
    )(ea_p, We1T, be1.reshape(1, 128))


def _msg_body(h_ref, xj_ref, w2t_ref, b2_ref, s_ref, o_ref):
    w = jnp.dot(h_ref[...], w2t_ref[...],
                preferred_element_type=jnp.float32) + b2_ref[...]
    xt = jnp.tile(xj_ref[...], (1, DIM))
    o_ref[...] = jnp.dot(w * xt, s_ref[...], preferred_element_type=jnp.float32)


def _messages(h_e, xj, We2pT, be2p, S):
    return pl.pallas_call(
        _msg_body,
        grid=(EP // EB,),
        in_specs=[
            pl.BlockSpec((EB, 128), lambda i: (i, 0)),
            pl.BlockSpec((EB, DIM), lambda i: (i, 0)),
            pl.BlockSpec((128, DIM * DIM), lambda i: (0, 0)),
            pl.BlockSpec((1, DIM * DIM), lambda i: (0, 0)),
            pl.BlockSpec((DIM * DIM, DIM), lambda i: (0, 0)),
        ],
        out_specs=pl.BlockSpec((EB, DIM), lambda i: (i, 0)),
        out_shape=jax.ShapeDtypeStruct((EP, DIM), jnp.float32),
    )(h_e, xj, We2pT, be2p, S)


def _update_body(p_ref, c_ref, x_ref, h_ref, wroot_ref, bconv_ref,
                 wih_ref, bih_ref, whh_ref, bhh_ref, o_ref):
    s = p_ref[0] + p_ref[1]
    cnt = c_ref[0][:, :1] + c_ref[1][:, :1]
    mean = s / jnp.maximum(cnt, 1.0)
    conv = (jnp.dot(x_ref[...], wroot_ref[...],
                    preferred_element_type=jnp.float32) + mean + bconv_ref[...])
    m = _leaky(conv)
    gi = jnp.dot(m, wih_ref[...], preferred_element_type=jnp.float32) + bih_ref[...]
    gh = jnp.dot(h_ref[...], whh_ref[...], preferred_element_type=jnp.float32) + bhh_ref[...]
    r = jax.nn.sigmoid(gi[:, :DIM] + gh[:, :DIM])
    z = jax.nn.sigmoid(gi[:, DIM:2 * DIM] + gh[:, DIM:2 * DIM])
    n = jnp.tanh(gi[:, 2 * DIM:] + r * gh[:, 2 * DIM:])
    hn = (1.0 - z) * n + z * h_ref[...]
    row = lax.broadcasted_iota(jnp.int32, (NP, 1), 0)
    o_ref[...] = jnp.where(row < N, hn, 0.0)


def _update(parts, cparts, x_cur, h_prev, WrootT, bconv, W_ihT, b_ih, W_hhT, b_hh):
    return pl.pallas_call(
        _update_body,
        out_shape=jax.ShapeDtypeStruct((NP, DIM), jnp.float32),
    )(parts, cparts, x_cur, h_prev, WrootT, bconv.reshape(1, DIM),
      W_ihT, b_ih.reshape(1, 3 * DIM), W_hhT, b_hh.reshape(1, 3 * DIM))


def _s2s_body(x_ref, b_ref, wlih_ref, blih_ref, wlhh_ref, blhh_ref,
              w1t_ref, b1_ref, o_ref, e_ref):
    q_star = jnp.zeros((B, 2 * DIM), jnp.float32)
    hx = jnp.zeros((B, DIM), jnp.float32)
    cx = jnp.zeros((B, DIM), jnp.float32)
    for _ in range(3):
        gates = (jnp.dot(q_star, wlih_ref[...], preferred_element_type=jnp.float32)
                 + blih_ref[...]
                 + jnp.dot(hx, wlhh_ref[...], preferred_element_type=jnp.float32)
                 + blhh_ref[...])
        i_g = jax.nn.sigmoid(gates[:, :DIM])
        f_g = jax.nn.sigmoid(gates[:, DIM:2 * DIM])
        g_g = jnp.tanh(gates[:, 2 * DIM:3 * DIM])
        o_g = jax.nn.sigmoid(gates[:, 3 * DIM:])
        cx = f_g * cx + i_g * g_g
        hx = o_g * jnp.tanh(cx)
        q = hx

        gid = lax.broadcasted_iota(jnp.int32, (CH, B), 1)
        emax = jnp.full((B,), -1e30, jnp.float32)
        for c in range(NC):
            x_c = x_ref[c * CH:(c + 1) * CH, :]
            b_c = b_ref[c * CH:(c + 1) * CH, :]
            mk = (b_c == gid)
            mf = mk.astype(jnp.float32)
            qg = jnp.dot(mf, q, preferred_element_type=jnp.float32)
            e_c = jnp.sum(x_c * qg, axis=1)
            em = jnp.where(mk, e_c[:, None], -1e30)
            emax = jnp.maximum(emax, jnp.max(em, axis=0))
            e_ref[c, :] = e_c
        denom = jnp.zeros((B,), jnp.float32)
        racc = jnp.zeros((B, DIM), jnp.float32)
        for c in range(NC):
            x_c = x_ref[c * CH:(c + 1) * CH, :]
            b_c = b_ref[c * CH:(c + 1) * CH, :]
            mf = (b_c == gid).astype(jnp.float32)
            me = jnp.sum(mf * emax[None, :], axis=1)
            ex = jnp.exp(e_ref[c, :] - me)
            denom = denom + jnp.dot(ex, mf, preferred_element_type=jnp.float32)
            racc = racc + lax.dot_general(mf, ex[:, None] * x_c,
                                          (((0,), (0,)), ((), ())),
                                          preferred_element_type=jnp.float32)
        r = racc / jnp.maximum(denom, 1e-30)[:, None]
        q_star = jnp.concatenate([q, r], axis=1)
    o_ref[...] = jnp.dot(q_star, w1t_ref[...],
                         preferred_element_type=jnp.float32) + b1_ref[...]


def _set2set(x_nodes, batch_p, Wl_ihT, bl_ih, Wl_hhT, bl_hh, W1T, b1):
    return pl.pallas_call(
        _s2s_body,
        out_shape=jax.ShapeDtypeStruct((B, 1), jnp.float32),
        scratch_shapes=[pltpu.VMEM((NC, CH), jnp.float32)],
    )(x_nodes, batch_p, Wl_ihT, bl_ih.reshape(1, 4 * DIM),
      Wl_hhT, bl_hh.reshape(1, 4 * DIM), W1T, b1.reshape(1, 1))


NW = 32
NJ = EP // NW // 128
RPT = NP // 16

_SC_MESH = plsc.VectorSubcoreMesh(core_axis_name="c", subcore_axis_name="s")
_SC_PARAMS = pltpu.CompilerParams(use_tc_tiling_on_sc=False)


def _sc_gather_body(x_hbm, idx_hbm, out_hbm, idx_v, rows_v, sem):
    wid = lax.axis_index("c") * 16 + lax.axis_index("s")
    pltpu.sync_copy(idx_hbm.at[wid], idx_v)

    @pl.loop(0, NJ)
    def _(j):
        pltpu.async_copy(x_hbm.at[idx_v.at[j]], rows_v, sem).wait()
        pltpu.sync_copy(rows_v, out_hbm.at[wid, j])


def _sc_gather(x_cur, idx_r):
    fn = pl.kernel(
        _sc_gather_body,
        out_type=jax.ShapeDtypeStruct((NW, NJ, 128, DIM), jnp.float32),
        mesh=_SC_MESH,
        compiler_params=_SC_PARAMS,
        scratch_types=[
            pltpu.VMEM((NJ, 128), jnp.int32),
            pltpu.VMEM((128, DIM), jnp.float32),
            pltpu.SemaphoreType.DMA,
        ],
    )
    return fn(x_cur, idx_r).reshape(EP, DIM)


def _sc_scatter_body(msg_hbm, idx_hbm, zero_hbm, out_hbm, idx_v, row_v, acc, sem):
    cid = lax.axis_index("c")
    sid = lax.axis_index("s")
    wid = cid * 16 + sid
    pltpu.sync_copy(zero_hbm.at[pl.ds(sid * RPT, RPT)],
                    acc.at[pl.ds(sid * RPT, RPT)])
    plsc.subcore_barrier()
    pltpu.sync_copy(idx_hbm.at[wid], idx_v)

    @pl.loop(0, NJ)
    def _(j):
        pltpu.sync_copy(msg_hbm.at[wid, j], row_v)
        pltpu.sync_copy(row_v, acc.at[idx_v.at[j]], add=True)

    plsc.subcore_barrier()
    pltpu.sync_copy(acc.at[pl.ds(sid * RPT, RPT)],
                    out_hbm.at[cid, pl.ds(sid * RPT, RPT)])


def _sc_scatter(msg, idx_r, zero):
    fn = pl.kernel(
        _sc_scatter_body,
        out_type=jax.ShapeDtypeStruct((2, NP, DIM), jnp.float32),
        mesh=_SC_MESH,
        compiler_params=_SC_PARAMS,
        scratch_types=[
            pltpu.VMEM((NJ, 128), jnp.int32),
            pltpu.VMEM((128, DIM), jnp.float32),
            pltpu.VMEM_SHARED((NP, DIM), jnp.float32),
            pltpu.SemaphoreType.DMA,
        ],
    )
    return fn(msg.reshape(NW, NJ, 128, DIM), idx_r, zero)


def kernel(x, edge_index, edge_attr, batch, W0, b0, We1, be1, We2, be2,
           Wroot, bconv, W_ih, W_hh, b_ih, b_hh, Wl_ih, Wl_hh, bl_ih, bl_hh,
           W1, b1):
    f32 = jnp.float32
    x_p = jnp.zeros((NP, NUM_FEAT), f32).at[:N].set(x)
    src_p = jnp.full((EP,), N, jnp.int32).at[:E].set(edge_index[0])
    dst_p = jnp.zeros((EP,), jnp.int32).at[:E].set(edge_index[1])
    ea_p = jnp.zeros((EP, 4), f32).at[:E].set(edge_attr)
    batch_p = jnp.full((NP, 1), B, jnp.int32).at[:N, 0].set(batch)

    We2p = We2.reshape(DIM, DIM, 128).transpose(1, 0, 2).reshape(DIM * DIM, 128)
    be2p = be2.reshape(DIM, DIM).T.reshape(1, DIM * DIM)
    S = jnp.repeat(jnp.eye(DIM, dtype=f32), DIM, axis=0)

    ones_msk = (jnp.arange(EP) < E).astype(f32)[:, None] * jnp.ones((1, DIM), f32)

    src_r = src_p.reshape(NW, NJ, 128)
    dst_r = dst_p.reshape(NW, NJ, 128)
    zero = jnp.zeros((NP, DIM), f32)

    out = _encode(x_p, W0.T, b0)
    h_e = _edge_mlp(ea_p, We1.T, be1)
    We2pT_bf = We2p.T.astype(jnp.bfloat16)

    cparts = _sc_scatter(ones_msk, dst_r, zero)

    h = out
    for _ in range(2):
        xj = _sc_gather(out, src_r)
        msg = _messages(h_e, xj, We2pT_bf, be2p, S)
        parts = _sc_scatter(msg, dst_r, zero)
        h = _update(parts, cparts, out, h, Wroot, bconv,
                    W_ih.T, b_ih, W_hh.T, b_hh)
        out = h

    return _set2set(out, batch_p, Wl_ih.T, bl_ih, Wl_hh.T, bl_hh, W1.T, b1)


# speedup vs baseline: 2.9539x; 1.1175x over previous
import functools

import jax
import jax.numpy as jnp
from jax import lax
from jax.experimental import pallas as pl
from jax.experimental.pallas import tpu as pltpu
from jax.experimental.pallas import tpu_sc as plsc

N = 10000
E = 160000
NUM_FEAT = 128
DIM = 32
B = 312
NEG_SLOPE = 0.01

NP = 10240
EP = 163840
EB = 1024
CH = 1280
NC = NP // CH


def _leaky(v):
    return jnp.where(v >= 0, v, NEG_SLOPE * v)


def _encode_body(x_ref, w0t_ref, b0_ref, o_ref):
    v = _leaky(jnp.dot(x_ref[...], w0t_ref[...],
                       preferred_element_type=jnp.float32) + b0_ref[...])
    row = lax.broadcasted_iota(jnp.int32, (NP, 1), 0)
    o_ref[...] = jnp.where(row < N, v, 0.0)


def _encode(x_p, W0T, b0):
    return pl.pallas_call(
        _encode_body,
        out_shape=jax.ShapeDtypeStruct((NP, DIM), jnp.float32),
    )(x_p, W0T, b0.reshape(1, DIM))


def _emlp_body(ea_ref, w1t_ref, b1_ref, o_ref):
    o_ref[...] = _leaky(jnp.dot(ea_ref[...], w1t_ref[...],
                                preferred_element_type=jnp.float32) + b1_ref[...])


def _edge_mlp(ea_p, We1T, be1):
    blk = 8192
    return pl.pallas_call(
        _emlp_body,
        grid=(EP // blk,),
        in_specs=[
            pl.BlockSpec((blk, 4), lambda i: (i, 0)),
            pl.BlockSpec((4, 128), lambda i: (0, 0)),
            pl.BlockSpec((1, 128), lambda i: (0, 0)),
        ],
        out_specs=pl.BlockSpec((blk, 128), lambda i: (i, 0)),
        out_shape=jax.ShapeDtypeStruct((EP, 128), jnp.float32),
    )(ea_p, We1T, be1.reshape(1, 128))


def _msg_body(h_ref, xj_ref, w2t_ref, b2_ref, s_ref, o_ref):
    w = jnp.dot(h_ref[...], w2t_ref[...],
                preferred_element_type=jnp.float32) + b2_ref[...]
    xt = jnp.tile(xj_ref[...], (1, DIM))
    o_ref[...] = jnp.dot(w * xt, s_ref[...], preferred_element_type=jnp.float32)


def _messages(h_e, xj, We2pT, be2p, S):
    return pl.pallas_call(
        _msg_body,
        grid=(EP // EB,),
        in_specs=[
            pl.BlockSpec((EB, 128), lambda i: (i, 0)),
            pl.BlockSpec((EB, DIM), lambda i: (i, 0)),
            pl.BlockSpec((128, DIM * DIM), lambda i: (0, 0)),
            pl.BlockSpec((1, DIM * DIM), lambda i: (0, 0)),
            pl.BlockSpec((DIM * DIM, DIM), lambda i: (0, 0)),
        ],
        out_specs=pl.BlockSpec((EB, DIM), lambda i: (i, 0)),
        out_shape=jax.ShapeDtypeStruct((EP, DIM), jnp.float32),
    )(h_e, xj, We2pT, be2p, S)


def _update_body(p_ref, c_ref, x_ref, h_ref, wroot_ref, bconv_ref,
                 wih_ref, bih_ref, whh_ref, bhh_ref, o_ref):
    s = p_ref[0] + p_ref[1]
    cnt = c_ref[0][:, :1] + c_ref[1][:, :1]
    mean = s / jnp.maximum(cnt, 1.0)
    conv = (jnp.dot(x_ref[...], wroot_ref[...],
                    preferred_element_type=jnp.float32) + mean + bconv_ref[...])
    m = _leaky(conv)
    gi = jnp.dot(m, wih_ref[...], preferred_element_type=jnp.float32) + bih_ref[...]
    gh = jnp.dot(h_ref[...], whh_ref[...], preferred_element_type=jnp.float32) + bhh_ref[...]
    r = jax.nn.sigmoid(gi[:, :DIM] + gh[:, :DIM])
    z = jax.nn.sigmoid(gi[:, DIM:2 * DIM] + gh[:, DIM:2 * DIM])
    n = jnp.tanh(gi[:, 2 * DIM:] + r * gh[:, 2 * DIM:])
    hn = (1.0 - z) * n + z * h_ref[...]
    row = lax.broadcasted_iota(jnp.int32, (NP, 1), 0)
    o_ref[...] = jnp.where(row < N, hn, 0.0)


def _update(parts, cparts, x_cur, h_prev, WrootT, bconv, W_ihT, b_ih, W_hhT, b_hh):
    return pl.pallas_call(
        _update_body,
        out_shape=jax.ShapeDtypeStruct((NP, DIM), jnp.float32),
    )(parts, cparts, x_cur, h_prev, WrootT, bconv.reshape(1, DIM),
      W_ihT, b_ih.reshape(1, 3 * DIM), W_hhT, b_hh.reshape(1, 3 * DIM))


def _s2s_body(x_ref, b_ref, wlih_ref, blih_ref, wlhh_ref, blhh_ref,
              w1t_ref, b1_ref, o_ref, e_ref):
    q_star = jnp.zeros((B, 2 * DIM), jnp.float32)
    hx = jnp.zeros((B, DIM), jnp.float32)
    cx = jnp.zeros((B, DIM), jnp.float32)
    for _ in range(3):
        gates = (jnp.dot(q_star, wlih_ref[...], preferred_element_type=jnp.float32)
                 + blih_ref[...]
                 + jnp.dot(hx, wlhh_ref[...], preferred_element_type=jnp.float32)
                 + blhh_ref[...])
        i_g = jax.nn.sigmoid(gates[:, :DIM])
        f_g = jax.nn.sigmoid(gates[:, DIM:2 * DIM])
        g_g = jnp.tanh(gates[:, 2 * DIM:3 * DIM])
        o_g = jax.nn.sigmoid(gates[:, 3 * DIM:])
        cx = f_g * cx + i_g * g_g
        hx = o_g * jnp.tanh(cx)
        q = hx

        gid = lax.broadcasted_iota(jnp.int32, (CH, B), 1)
        emax = jnp.full((B,), -1e30, jnp.float32)
        for c in range(NC):
            x_c = x_ref[c * CH:(c + 1) * CH, :]
            b_c = b_ref[c * CH:(c + 1) * CH, :]
            mk = (b_c == gid)
            mf = mk.astype(jnp.float32)
            qg = jnp.dot(mf, q, preferred_element_type=jnp.float32)
            e_c = jnp.sum(x_c * qg, axis=1)
            em = jnp.where(mk, e_c[:, None], -1e30)
            emax = jnp.maximum(emax, jnp.max(em, axis=0))
            e_ref[c, :] = e_c
        denom = jnp.zeros((B,), jnp.float32)
        racc = jnp.zeros((B, DIM), jnp.float32)
        for c in range(NC):
            x_c = x_ref[c * CH:(c + 1) * CH, :]
            b_c = b_ref[c * CH:(c + 1) * CH, :]
            mf = (b_c == gid).astype(jnp.float32)
            me = jnp.sum(mf * emax[None, :], axis=1)
            ex = jnp.exp(e_ref[c, :] - me)
            denom = denom + jnp.dot(ex, mf, preferred_element_type=jnp.float32)
            racc = racc + lax.dot_general(mf, ex[:, None] * x_c,
                                          (((0,), (0,)), ((), ())),
                                          preferred_element_type=jnp.float32)
        r = racc / jnp.maximum(denom, 1e-30)[:, None]
        q_star = jnp.concatenate([q, r], axis=1)
    o_ref[...] = jnp.dot(q_star, w1t_ref[...],
                         preferred_element_type=jnp.float32) + b1_ref[...]


def _set2set(x_nodes, batch_p, Wl_ihT, bl_ih, Wl_hhT, bl_hh, W1T, b1):
    return pl.pallas_call(
        _s2s_body,
        out_shape=jax.ShapeDtypeStruct((B, 1), jnp.float32),
        scratch_shapes=[pltpu.VMEM((NC, CH), jnp.float32)],
    )(x_nodes, batch_p, Wl_ihT, bl_ih.reshape(1, 4 * DIM),
      Wl_hhT, bl_hh.reshape(1, 4 * DIM), W1T, b1.reshape(1, 1))


NW = 32
NJ = EP // NW // 128
RPT = NP // 16

_SC_MESH = plsc.VectorSubcoreMesh(core_axis_name="c", subcore_axis_name="s")
_SC_PARAMS = pltpu.CompilerParams(use_tc_tiling_on_sc=False)


def _sc_gather_body(x_hbm, idx_hbm, out_hbm, idx_v, rows_v, sem):
    wid = lax.axis_index("c") * 16 + lax.axis_index("s")
    pltpu.sync_copy(idx_hbm.at[wid], idx_v)

    @pl.loop(0, NJ)
    def _(j):
        pltpu.async_copy(x_hbm.at[idx_v.at[j]], rows_v, sem).wait()
        pltpu.sync_copy(rows_v, out_hbm.at[wid, j])


def _sc_gather(x_cur, idx_r):
    fn = pl.kernel(
        _sc_gather_body,
        out_type=jax.ShapeDtypeStruct((NW, NJ, 128, DIM), jnp.float32),
        mesh=_SC_MESH,
        compiler_params=_SC_PARAMS,
        scratch_types=[
            pltpu.VMEM((NJ, 128), jnp.int32),
            pltpu.VMEM((128, DIM), jnp.float32),
            pltpu.SemaphoreType.DMA,
        ],
    )
    return fn(x_cur, idx_r).reshape(EP, DIM)


def _sc_scatter_body(msg_hbm, idx_hbm, zero_hbm, out_hbm, idx_v, row_v, acc, sem):
    cid = lax.axis_index("c")
    sid = lax.axis_index("s")
    wid = cid * 16 + sid
    pltpu.sync_copy(zero_hbm.at[pl.ds(sid * RPT, RPT)],
                    acc.at[pl.ds(sid * RPT, RPT)])
    plsc.subcore_barrier()
    pltpu.sync_copy(idx_hbm.at[wid], idx_v)

    @pl.loop(0, NJ)
    def _(j):
        pltpu.sync_copy(msg_hbm.at[wid, j], row_v)
        pltpu.sync_copy(row_v, acc.at[idx_v.at[j]], add=True)

    plsc.subcore_barrier()
    pltpu.sync_copy(acc.at[pl.ds(sid * RPT, RPT)],
                    out_hbm.at[cid, pl.ds(sid * RPT, RPT)])


def _sc_scatter(msg, idx_r, zero):
    fn = pl.kernel(
        _sc_scatter_body,
        out_type=jax.ShapeDtypeStruct((2, NP, DIM), jnp.float32),
        mesh=_SC_MESH,
        compiler_params=_SC_PARAMS,
        scratch_types=[
            pltpu.VMEM((NJ, 128), jnp.int32),
            pltpu.VMEM((128, DIM), jnp.float32),
            pltpu.VMEM_SHARED((NP, DIM), jnp.float32),
            pltpu.SemaphoreType.DMA,
        ],
    )
    return fn(msg.reshape(NW, NJ, 128, DIM), idx_r, zero)


def kernel(x, edge_index, edge_attr, batch, W0, b0, We1, be1, We2, be2,
           Wroot, bconv, W_ih, W_hh, b_ih, b_hh, Wl_ih, Wl_hh, bl_ih, bl_hh,
           W1, b1):
    f32 = jnp.float32
    x_p = jnp.zeros((NP, NUM_FEAT), f32).at[:N].set(x)
    src_p = jnp.full((EP,), N, jnp.int32).at[:E].set(edge_index[0])
    dst_p = jnp.zeros((EP,), jnp.int32).at[:E].set(edge_index[1])
    ea_p = jnp.zeros((EP, 4), f32).at[:E].set(edge_attr)
    batch_p = jnp.full((NP, 1), B, jnp.int32).at[:N, 0].set(batch)

    We2p = We2.reshape(DIM, DIM, 128).transpose(1, 0, 2).reshape(DIM * DIM, 128)
    be2p = be2.reshape(DIM, DIM).T.reshape(1, DIM * DIM)
    S = jnp.repeat(jnp.eye(DIM, dtype=f32), DIM, axis=0)

    ones_msk = (jnp.arange(EP) < E).astype(f32)[:, None] * jnp.ones((1, DIM), f32)

    src_r = src_p.reshape(NW, NJ, 128)
    dst_r = dst_p.reshape(NW, NJ, 128)
    zero = jnp.zeros((NP, DIM), f32)

    out = _encode(x_p, W0.T, b0)
    h_e = _edge_mlp(ea_p, We1.T, be1)

    cparts = _sc_scatter(ones_msk, dst_r, zero)

    h = out
    for _ in range(2):
        xj = _sc_gather(out, src_r)
        msg = _messages(h_e, xj, We2p.T, be2p, S)
        parts = _sc_scatter(msg, dst_r, zero)
        h = _update(parts, cparts, out, h, Wroot, bconv,
                    W_ih.T, b_ih, W_hh.T, b_hh)
        out = h

    return _set2set(out, batch_p, Wl_ih.T, bl_ih, Wl_hh.T, bl_hh, W1.T, b1)


# speedup vs baseline: 3.1185x; 1.0557x over previous
import functools

import jax
import jax.numpy as jnp
from jax import lax
from jax.experimental import pallas as pl
from jax.experimental.pallas import tpu as pltpu
from jax.experimental.pallas import tpu_sc as plsc

N = 10000
E = 160000
NUM_FEAT = 128
DIM = 32
B = 312
NEG_SLOPE = 0.01

NP = 10240
EP = 163840
EB = 2048
CH = 1280
NC = NP // CH


def _leaky(v):
    return jnp.where(v >= 0, v, NEG_SLOPE * v)


def _encode_body(x_ref, w0t_ref, b0_ref, o_ref):
    v = _leaky(jnp.dot(x_ref[...], w0t_ref[...],
                       preferred_element_type=jnp.float32) + b0_ref[...])
    row = lax.broadcasted_iota(jnp.int32, (NP, 1), 0)
    o_ref[...] = jnp.where(row < N, v, 0.0)


def _encode(x_p, W0T, b0):
    return pl.pallas_call(
        _encode_body,
        out_shape=jax.ShapeDtypeStruct((NP, DIM), jnp.float32),
    )(x_p, W0T, b0.reshape(1, DIM))


def _emlp_body(ea_ref, w1t_ref, b1_ref, o_ref):
    o_ref[...] = _leaky(jnp.dot(ea_ref[...], w1t_ref[...],
                                preferred_element_type=jnp.float32) + b1_ref[...])


def _edge_mlp(ea_p, We1T, be1):
    blk = 8192
    return pl.pallas_call(
        _emlp_body,
        grid=(EP // blk,),
        in_specs=[
            pl.BlockSpec((blk, 4), lambda i: (i, 0)),
            pl.BlockSpec((4, 128), lambda i: (0, 0)),
            pl.BlockSpec((1, 128), lambda i: (0, 0)),
        ],
        out_specs=pl.BlockSpec((blk, 128), lambda i: (i, 0)),
        out_shape=jax.ShapeDtypeStruct((EP, 128), jnp.float32),
    )(ea_p, We1T, be1.reshape(1, 128))


def _msg_body(h_ref, xj_ref, w2t_ref, b2_ref, s_ref, o_ref):
    w = jnp.dot(h_ref[...], w2t_ref[...],
                preferred_element_type=jnp.float32) + b2_ref[...]
    xt = jnp.tile(xj_ref[...], (1, DIM))
    o_ref[...] = jnp.dot(w * xt, s_ref[...], preferred_element_type=jnp.float32)


def _messages(h_e, xj, We2pT, be2p, S):
    return pl.pallas_call(
        _msg_body,
        grid=(EP // EB,),
        in_specs=[
            pl.BlockSpec((EB, 128), lambda i: (i, 0)),
            pl.BlockSpec((EB, DIM), lambda i: (i, 0)),
            pl.BlockSpec((128, DIM * DIM), lambda i: (0, 0)),
            pl.BlockSpec((1, DIM * DIM), lambda i: (0, 0)),
            pl.BlockSpec((DIM * DIM, DIM), lambda i: (0, 0)),
        ],
        out_specs=pl.BlockSpec((EB, DIM), lambda i: (i, 0)),
        out_shape=jax.ShapeDtypeStruct((EP, DIM), jnp.float32),
    )(h_e, xj, We2pT, be2p, S)


def _update_body(p_ref, c_ref, x_ref, h_ref, wroot_ref, bconv_ref,
                 wih_ref, bih_ref, whh_ref, bhh_ref, o_ref):
    s = p_ref[0] + p_ref[1]
    cnt = c_ref[0][:, :1] + c_ref[1][:, :1]
    mean = s / jnp.maximum(cnt, 1.0)
    conv = (jnp.dot(x_ref[...], wroot_ref[...],
                    preferred_element_type=jnp.float32) + mean + bconv_ref[...])
    m = _leaky(conv)
    gi = jnp.dot(m, wih_ref[...], preferred_element_type=jnp.float32) + bih_ref[...]
    gh = jnp.dot(h_ref[...], whh_ref[...], preferred_element_type=jnp.float32) + bhh_ref[...]
    r = jax.nn.sigmoid(gi[:, :DIM] + gh[:, :DIM])
    z = jax.nn.sigmoid(gi[:, DIM:2 * DIM] + gh[:, DIM:2 * DIM])
    n = jnp.tanh(gi[:, 2 * DIM:] + r * gh[:, 2 * DIM:])
    hn = (1.0 - z) * n + z * h_ref[...]
    row = lax.broadcasted_iota(jnp.int32, (NP, 1), 0)
    o_ref[...] = jnp.where(row < N, hn, 0.0)


def _update(parts, cparts, x_cur, h_prev, WrootT, bconv, W_ihT, b_ih, W_hhT, b_hh):
    return pl.pallas_call(
        _update_body,
        out_shape=jax.ShapeDtypeStruct((NP, DIM), jnp.float32),
    )(parts, cparts, x_cur, h_prev, WrootT, bconv.reshape(1, DIM),
      W_ihT, b_ih.reshape(1, 3 * DIM), W_hhT, b_hh.reshape(1, 3 * DIM))


def _s2s_body(x_ref, b_ref, wlih_ref, blih_ref, wlhh_ref, blhh_ref,
              w1t_ref, b1_ref, o_ref, e_ref):
    q_star = jnp.zeros((B, 2 * DIM), jnp.float32)
    hx = jnp.zeros((B, DIM), jnp.float32)
    cx = jnp.zeros((B, DIM), jnp.float32)
    for _ in range(3):
        gates = (jnp.dot(q_star, wlih_ref[...], preferred_element_type=jnp.float32)
                 + blih_ref[...]
                 + jnp.dot(hx, wlhh_ref[...], preferred_element_type=jnp.float32)
                 + blhh_ref[...])
        i_g = jax.nn.sigmoid(gates[:, :DIM])
        f_g = jax.nn.sigmoid(gates[:, DIM:2 * DIM])
        g_g = jnp.tanh(gates[:, 2 * DIM:3 * DIM])
        o_g = jax.nn.sigmoid(gates[:, 3 * DIM:])
        cx = f_g * cx + i_g * g_g
        hx = o_g * jnp.tanh(cx)
        q = hx

        gid = lax.broadcasted_iota(jnp.int32, (CH, B), 1)
        emax = jnp.full((B,), -1e30, jnp.float32)
        for c in range(NC):
            x_c = x_ref[c * CH:(c + 1) * CH, :]
            b_c = b_ref[c * CH:(c + 1) * CH, :]
            mk = (b_c == gid)
            mf = mk.astype(jnp.float32)
            qg = jnp.dot(mf, q, preferred_element_type=jnp.float32)
            e_c = jnp.sum(x_c * qg, axis=1)
            em = jnp.where(mk, e_c[:, None], -1e30)
            emax = jnp.maximum(emax, jnp.max(em, axis=0))
            e_ref[c, :] = e_c
        denom = jnp.zeros((B,), jnp.float32)
        racc = jnp.zeros((B, DIM), jnp.float32)
        for c in range(NC):
            x_c = x_ref[c * CH:(c + 1) * CH, :]
            b_c = b_ref[c * CH:(c + 1) * CH, :]
            mf = (b_c == gid).astype(jnp.float32)
            me = jnp.sum(mf * emax[None, :], axis=1)
            ex = jnp.exp(e_ref[c, :] - me)
            denom = denom + jnp.dot(ex, mf, preferred_element_type=jnp.float32)
            racc = racc + lax.dot_general(mf, ex[:, None] * x_c,
                                          (((0,), (0,)), ((), ())),
                                          preferred_element_type=jnp.float32)
        r = racc / jnp.maximum(denom, 1e-30)[:, None]
        q_star = jnp.concatenate([q, r], axis=1)
    o_ref[...] = jnp.dot(q_star, w1t_ref[...],
                         preferred_element_type=jnp.float32) + b1_ref[...]


def _set2set(x_nodes, batch_p, Wl_ihT, bl_ih, Wl_hhT, bl_hh, W1T, b1):
    return pl.pallas_call(
        _s2s_body,
        out_shape=jax.ShapeDtypeStruct((B, 1), jnp.float32),
        scratch_shapes=[pltpu.VMEM((NC, CH), jnp.float32)],
    )(x_nodes, batch_p, Wl_ihT, bl_ih.reshape(1, 4 * DIM),
      Wl_hhT, bl_hh.reshape(1, 4 * DIM), W1T, b1.reshape(1, 1))


NW = 32
NJ = EP // NW // 128
RPT = NP // 16

_SC_MESH = plsc.VectorSubcoreMesh(core_axis_name="c", subcore_axis_name="s")
_SC_PARAMS = pltpu.CompilerParams(use_tc_tiling_on_sc=False)


def _sc_gather_body(x_hbm, idx_hbm, out_hbm, idx_v, rows_v, sem):
    wid = lax.axis_index("c") * 16 + lax.axis_index("s")
    pltpu.sync_copy(idx_hbm.at[wid], idx_v)

    @pl.loop(0, NJ)
    def _(j):
        pltpu.async_copy(x_hbm.at[idx_v.at[j]], rows_v, sem).wait()
        pltpu.sync_copy(rows_v, out_hbm.at[wid, j])


def _sc_gather(x_cur, idx_r):
    fn = pl.kernel(
        _sc_gather_body,
        out_type=jax.ShapeDtypeStruct((NW, NJ, 128, DIM), jnp.float32),
        mesh=_SC_MESH,
        compiler_params=_SC_PARAMS,
        scratch_types=[
            pltpu.VMEM((NJ, 128), jnp.int32),
            pltpu.VMEM((128, DIM), jnp.float32),
            pltpu.SemaphoreType.DMA,
        ],
    )
    return fn(x_cur, idx_r).reshape(EP, DIM)


def _sc_scatter_body(msg_hbm, idx_hbm, zero_hbm, out_hbm, idx_v, row_v, acc, sem):
    cid = lax.axis_index("c")
    sid = lax.axis_index("s")
    wid = cid * 16 + sid
    pltpu.sync_copy(zero_hbm.at[pl.ds(sid * RPT, RPT)],
                    acc.at[pl.ds(sid * RPT, RPT)])
    plsc.subcore_barrier()
    pltpu.sync_copy(idx_hbm.at[wid], idx_v)

    @pl.loop(0, NJ)
    def _(j):
        pltpu.sync_copy(msg_hbm.at[wid, j], row_v)
        pltpu.sync_copy(row_v, acc.at[idx_v.at[j]], add=True)

    plsc.subcore_barrier()
    pltpu.sync_copy(acc.at[pl.ds(sid * RPT, RPT)],
                    out_hbm.at[cid, pl.ds(sid * RPT, RPT)])


def _sc_scatter(msg, idx_r, zero):
    fn = pl.kernel(
        _sc_scatter_body,
        out_type=jax.ShapeDtypeStruct((2, NP, DIM), jnp.float32),
        mesh=_SC_MESH,
        compiler_params=_SC_PARAMS,
        scratch_types=[
            pltpu.VMEM((NJ, 128), jnp.int32),
            pltpu.VMEM((128, DIM), jnp.float32),
            pltpu.VMEM_SHARED((NP, DIM), jnp.float32),
            pltpu.SemaphoreType.DMA,
        ],
    )
    return fn(msg.reshape(NW, NJ, 128, DIM), idx_r, zero)


def kernel(x, edge_index, edge_attr, batch, W0, b0, We1, be1, We2, be2,
           Wroot, bconv, W_ih, W_hh, b_ih, b_hh, Wl_ih, Wl_hh, bl_ih, bl_hh,
           W1, b1):
    f32 = jnp.float32
    x_p = jnp.zeros((NP, NUM_FEAT), f32).at[:N].set(x)
    src_p = jnp.full((EP,), N, jnp.int32).at[:E].set(edge_index[0])
    dst_p = jnp.zeros((EP,), jnp.int32).at[:E].set(edge_index[1])
    ea_p = jnp.zeros((EP, 4), f32).at[:E].set(edge_attr)
    batch_p = jnp.full((NP, 1), B, jnp.int32).at[:N, 0].set(batch)

    We2p = We2.reshape(DIM, DIM, 128).transpose(1, 0, 2).reshape(DIM * DIM, 128)
    be2p = be2.reshape(DIM, DIM).T.reshape(1, DIM * DIM)
    S = jnp.repeat(jnp.eye(DIM, dtype=f32), DIM, axis=0)

    ones_msk = (jnp.arange(EP) < E).astype(f32)[:, None] * jnp.ones((1, DIM), f32)

    src_r = src_p.reshape(NW, NJ, 128)
    dst_r = dst_p.reshape(NW, NJ, 128)
    zero = jnp.zeros((NP, DIM), f32)

    out = _encode(x_p, W0.T, b0)
    h_e = _edge_mlp(ea_p, We1.T, be1)

    cparts = _sc_scatter(ones_msk, dst_r, zero)

    h = out
    for _ in range(2):
        xj = _sc_gather(out, src_r)
        msg = _messages(h_e, xj, We2p.T, be2p, S)
        parts = _sc_scatter(msg, dst_r, zero)
        h = _update(parts, cparts, out, h, Wroot, bconv,
                    W_ih.T, b_ih, W_hh.T, b_hh)
        out = h

    return _set2set(out, batch_p, Wl_ih.T, bl_ih, Wl_hh.T, bl_hh, W1.T, b1)


# speedup vs baseline: 3.1901x; 1.0230x over previous
import functools

import jax
import jax.numpy as jnp
from jax import lax
from jax.experimental import pallas as pl
from jax.experimental.pallas import tpu as pltpu
from jax.experimental.pallas import tpu_sc as plsc

N = 10000
E = 160000
NUM_FEAT = 128
DIM = 32
B = 312
NEG_SLOPE = 0.01

NP = 10240
EP = 163840
EB = 4096
CH = 1280
NC = NP // CH


def _leaky(v):
    return jnp.where(v >= 0, v, NEG_SLOPE * v)


def _encode_body(x_ref, w0t_ref, b0_ref, o_ref):
    v = _leaky(jnp.dot(x_ref[...], w0t_ref[...],
                       preferred_element_type=jnp.float32) + b0_ref[...])
    row = lax.broadcasted_iota(jnp.int32, (NP, 1), 0)
    o_ref[...] = jnp.where(row < N, v, 0.0)


def _encode(x_p, W0T, b0):
    return pl.pallas_call(
        _encode_body,
        out_shape=jax.ShapeDtypeStruct((NP, DIM), jnp.float32),
    )(x_p, W0T, b0.reshape(1, DIM))


def _emlp_body(ea_ref, w1t_ref, b1_ref, o_ref):
    o_ref[...] = _leaky(jnp.dot(ea_ref[...], w1t_ref[...],
                                preferred_element_type=jnp.float32) + b1_ref[...])


def _edge_mlp(ea_p, We1T, be1):
    blk = 8192
    return pl.pallas_call(
        _emlp_body,
        grid=(EP // blk,),
        in_specs=[
            pl.BlockSpec((blk, 4), lambda i: (i, 0)),
            pl.BlockSpec((4, 128), lambda i: (0, 0)),
            pl.BlockSpec((1, 128), lambda i: (0, 0)),
        ],
        out_specs=pl.BlockSpec((blk, 128), lambda i: (i, 0)),
        out_shape=jax.ShapeDtypeStruct((EP, 128), jnp.float32),
    )(ea_p, We1T, be1.reshape(1, 128))


def _msg_body(h_ref, xj_ref, w2t_ref, b2_ref, s_ref, o_ref):
    w = jnp.dot(h_ref[...], w2t_ref[...],
                preferred_element_type=jnp.float32) + b2_ref[...]
    xt = jnp.tile(xj_ref[...], (1, DIM))
    o_ref[...] = jnp.dot(w * xt, s_ref[...], preferred_element_type=jnp.float32)


def _messages(h_e, xj, We2pT, be2p, S):
    return pl.pallas_call(
        _msg_body,
        grid=(EP // EB,),
        in_specs=[
            pl.BlockSpec((EB, 128), lambda i: (i, 0)),
            pl.BlockSpec((EB, DIM), lambda i: (i, 0)),
            pl.BlockSpec((128, DIM * DIM), lambda i: (0, 0)),
            pl.BlockSpec((1, DIM * DIM), lambda i: (0, 0)),
            pl.BlockSpec((DIM * DIM, DIM), lambda i: (0, 0)),
        ],
        out_specs=pl.BlockSpec((EB, DIM), lambda i: (i, 0)),
        out_shape=jax.ShapeDtypeStruct((EP, DIM), jnp.float32),
    )(h_e, xj, We2pT, be2p, S)


def _update_body(p_ref, c_ref, x_ref, h_ref, wroot_ref, bconv_ref,
                 wih_ref, bih_ref, whh_ref, bhh_ref, o_ref):
    s = p_ref[0] + p_ref[1]
    cnt = c_ref[0][:, :1] + c_ref[1][:, :1]
    mean = s / jnp.maximum(cnt, 1.0)
    conv = (jnp.dot(x_ref[...], wroot_ref[...],
                    preferred_element_type=jnp.float32) + mean + bconv_ref[...])
    m = _leaky(conv)
    gi = jnp.dot(m, wih_ref[...], preferred_element_type=jnp.float32) + bih_ref[...]
    gh = jnp.dot(h_ref[...], whh_ref[...], preferred_element_type=jnp.float32) + bhh_ref[...]
    r = jax.nn.sigmoid(gi[:, :DIM] + gh[:, :DIM])
    z = jax.nn.sigmoid(gi[:, DIM:2 * DIM] + gh[:, DIM:2 * DIM])
    n = jnp.tanh(gi[:, 2 * DIM:] + r * gh[:, 2 * DIM:])
    hn = (1.0 - z) * n + z * h_ref[...]
    row = lax.broadcasted_iota(jnp.int32, (NP, 1), 0)
    o_ref[...] = jnp.where(row < N, hn, 0.0)


def _update(parts, cparts, x_cur, h_prev, WrootT, bconv, W_ihT, b_ih, W_hhT, b_hh):
    return pl.pallas_call(
        _update_body,
        out_shape=jax.ShapeDtypeStruct((NP, DIM), jnp.float32),
    )(parts, cparts, x_cur, h_prev, WrootT, bconv.reshape(1, DIM),
      W_ihT, b_ih.reshape(1, 3 * DIM), W_hhT, b_hh.reshape(1, 3 * DIM))


def _s2s_body(x_ref, b_ref, wlih_ref, blih_ref, wlhh_ref, blhh_ref,
              w1t_ref, b1_ref, o_ref, e_ref):
    q_star = jnp.zeros((B, 2 * DIM), jnp.float32)
    hx = jnp.zeros((B, DIM), jnp.float32)
    cx = jnp.zeros((B, DIM), jnp.float32)
    for _ in range(3):
        gates = (jnp.dot(q_star, wlih_ref[...], preferred_element_type=jnp.float32)
                 + blih_ref[...]
                 + jnp.dot(hx, wlhh_ref[...], preferred_element_type=jnp.float32)
                 + blhh_ref[...])
        i_g = jax.nn.sigmoid(gates[:, :DIM])
        f_g = jax.nn.sigmoid(gates[:, DIM:2 * DIM])
        g_g = jnp.tanh(gates[:, 2 * DIM:3 * DIM])
        o_g = jax.nn.sigmoid(gates[:, 3 * DIM:])
        cx = f_g * cx + i_g * g_g
        hx = o_g * jnp.tanh(cx)
        q = hx

        gid = lax.broadcasted_iota(jnp.int32, (CH, B), 1)
        emax = jnp.full((B,), -1e30, jnp.float32)
        for c in range(NC):
            x_c = x_ref[c * CH:(c + 1) * CH, :]
            b_c = b_ref[c * CH:(c + 1) * CH, :]
            mk = (b_c == gid)
            mf = mk.astype(jnp.float32)
            qg = jnp.dot(mf, q, preferred_element_type=jnp.float32)
            e_c = jnp.sum(x_c * qg, axis=1)
            em = jnp.where(mk, e_c[:, None], -1e30)
            emax = jnp.maximum(emax, jnp.max(em, axis=0))
            e_ref[c, :] = e_c
        denom = jnp.zeros((B,), jnp.float32)
        racc = jnp.zeros((B, DIM), jnp.float32)
        for c in range(NC):
            x_c = x_ref[c * CH:(c + 1) * CH, :]
            b_c = b_ref[c * CH:(c + 1) * CH, :]
            mf = (b_c == gid).astype(jnp.float32)
            me = jnp.sum(mf * emax[None, :], axis=1)
            ex = jnp.exp(e_ref[c, :] - me)
            denom = denom + jnp.dot(ex, mf, preferred_element_type=jnp.float32)
            racc = racc + lax.dot_general(mf, ex[:, None] * x_c,
                                          (((0,), (0,)), ((), ())),
                                          preferred_element_type=jnp.float32)
        r = racc / jnp.maximum(denom, 1e-30)[:, None]
        q_star = jnp.concatenate([q, r], axis=1)
    o_ref[...] = jnp.dot(q_star, w1t_ref[...],
                         preferred_element_type=jnp.float32) + b1_ref[...]


def _set2set(x_nodes, batch_p, Wl_ihT, bl_ih, Wl_hhT, bl_hh, W1T, b1):
    return pl.pallas_call(
        _s2s_body,
        out_shape=jax.ShapeDtypeStruct((B, 1), jnp.float32),
        scratch_shapes=[pltpu.VMEM((NC, CH), jnp.float32)],
    )(x_nodes, batch_p, Wl_ihT, bl_ih.reshape(1, 4 * DIM),
      Wl_hhT, bl_hh.reshape(1, 4 * DIM), W1T, b1.reshape(1, 1))


NW = 32
NJ = EP // NW // 128
RPT = NP // 16

_SC_MESH = plsc.VectorSubcoreMesh(core_axis_name="c", subcore_axis_name="s")
_SC_PARAMS = pltpu.CompilerParams(use_tc_tiling_on_sc=False)


def _sc_gather_body(x_hbm, idx_hbm, out_hbm, idx_v, rows_v, sem):
    wid = lax.axis_index("c") * 16 + lax.axis_index("s")
    pltpu.sync_copy(idx_hbm.at[wid], idx_v)

    @pl.loop(0, NJ)
    def _(j):
        pltpu.async_copy(x_hbm.at[idx_v.at[j]], rows_v, sem).wait()
        pltpu.sync_copy(rows_v, out_hbm.at[wid, j])


def _sc_gather(x_cur, idx_r):
    fn = pl.kernel(
        _sc_gather_body,
        out_type=jax.ShapeDtypeStruct((NW, NJ, 128, DIM), jnp.float32),
        mesh=_SC_MESH,
        compiler_params=_SC_PARAMS,
        scratch_types=[
            pltpu.VMEM((NJ, 128), jnp.int32),
            pltpu.VMEM((128, DIM), jnp.float32),
            pltpu.SemaphoreType.DMA,
        ],
    )
    return fn(x_cur, idx_r).reshape(EP, DIM)


def _sc_scatter_body(msg_hbm, idx_hbm, zero_hbm, out_hbm, idx_v, row_v, acc, sem):
    cid = lax.axis_index("c")
    sid = lax.axis_index("s")
    wid = cid * 16 + sid
    pltpu.sync_copy(zero_hbm.at[pl.ds(sid * RPT, RPT)],
                    acc.at[pl.ds(sid * RPT, RPT)])
    plsc.subcore_barrier()
    pltpu.sync_copy(idx_hbm.at[wid], idx_v)

    @pl.loop(0, NJ)
    def _(j):
        pltpu.sync_copy(msg_hbm.at[wid, j], row_v)
        pltpu.sync_copy(row_v, acc.at[idx_v.at[j]], add=True)

    plsc.subcore_barrier()
    pltpu.sync_copy(acc.at[pl.ds(sid * RPT, RPT)],
                    out_hbm.at[cid, pl.ds(sid * RPT, RPT)])


def _sc_scatter(msg, idx_r, zero):
    fn = pl.kernel(
        _sc_scatter_body,
        out_type=jax.ShapeDtypeStruct((2, NP, DIM), jnp.float32),
        mesh=_SC_MESH,
        compiler_params=_SC_PARAMS,
        scratch_types=[
            pltpu.VMEM((NJ, 128), jnp.int32),
            pltpu.VMEM((128, DIM), jnp.float32),
            pltpu.VMEM_SHARED((NP, DIM), jnp.float32),
            pltpu.SemaphoreType.DMA,
        ],
    )
    return fn(msg.reshape(NW, NJ, 128, DIM), idx_r, zero)


def kernel(x, edge_index, edge_attr, batch, W0, b0, We1, be1, We2, be2,
           Wroot, bconv, W_ih, W_hh, b_ih, b_hh, Wl_ih, Wl_hh, bl_ih, bl_hh,
           W1, b1):
    f32 = jnp.float32
    x_p = jnp.zeros((NP, NUM_FEAT), f32).at[:N].set(x)
    src_p = jnp.full((EP,), N, jnp.int32).at[:E].set(edge_index[0])
    dst_p = jnp.zeros((EP,), jnp.int32).at[:E].set(edge_index[1])
    ea_p = jnp.zeros((EP, 4), f32).at[:E].set(edge_attr)
    batch_p = jnp.full((NP, 1), B, jnp.int32).at[:N, 0].set(batch)

    We2p = We2.reshape(DIM, DIM, 128).transpose(1, 0, 2).reshape(DIM * DIM, 128)
    be2p = be2.reshape(DIM, DIM).T.reshape(1, DIM * DIM)
    S = jnp.repeat(jnp.eye(DIM, dtype=f32), DIM, axis=0)

    ones_msk = (jnp.arange(EP) < E).astype(f32)[:, None] * jnp.ones((1, DIM), f32)

    src_r = src_p.reshape(NW, NJ, 128)
    dst_r = dst_p.reshape(NW, NJ, 128)
    zero = jnp.zeros((NP, DIM), f32)

    out = _encode(x_p, W0.T, b0)
    h_e = _edge_mlp(ea_p, We1.T, be1)

    cparts = _sc_scatter(ones_msk, dst_r, zero)

    h = out
    for _ in range(2):
        xj = _sc_gather(out, src_r)
        msg = _messages(h_e, xj, We2p.T, be2p, S)
        parts = _sc_scatter(msg, dst_r, zero)
        h = _update(parts, cparts, out, h, Wroot, bconv,
                    W_ih.T, b_ih, W_hh.T, b_hh)
        out = h

    return _set2set(out, batch_p, Wl_ih.T, bl_ih, Wl_hh.T, bl_hh, W1.T, b1)


# speedup vs baseline: 3.2044x; 1.0045x over previous
import functools

import jax
import jax.numpy as jnp
from jax import lax
from jax.experimental import pallas as pl
from jax.experimental.pallas import tpu as pltpu
from jax.experimental.pallas import tpu_sc as plsc

N = 10000
E = 160000
NUM_FEAT = 128
DIM = 32
B = 312
NEG_SLOPE = 0.01

NP = 10240
EP = 163840
EB = 4096
CH = 1280
NC = NP // CH


def _leaky(v):
    return jnp.where(v >= 0, v, NEG_SLOPE * v)


def _encode_body(x_ref, w0t_ref, b0_ref, o_ref):
    v = _leaky(jnp.dot(x_ref[...], w0t_ref[...],
                       preferred_element_type=jnp.float32) + b0_ref[...])
    row = lax.broadcasted_iota(jnp.int32, (NP, 1), 0)
    o_ref[...] = jnp.where(row < N, v, 0.0)


def _encode(x_p, W0T, b0):
    return pl.pallas_call(
        _encode_body,
        out_shape=jax.ShapeDtypeStruct((NP, DIM), jnp.float32),
    )(x_p, W0T, b0.reshape(1, DIM))


def _msg_body(ea_ref, xj_ref, w1t_ref, b1_ref, w2t_ref, b2_ref, s_ref, o_ref):
    h = _leaky(jnp.dot(ea_ref[...], w1t_ref[...],
                       preferred_element_type=jnp.float32) + b1_ref[...])
    w = jnp.dot(h, w2t_ref[...], preferred_element_type=jnp.float32) + b2_ref[...]
    xt = jnp.tile(xj_ref[...], (1, DIM))
    o_ref[...] = jnp.dot(w * xt, s_ref[...], preferred_element_type=jnp.float32)


def _messages(ea_p, xj, We1T, be1, We2pT, be2p, S):
    return pl.pallas_call(
        _msg_body,
        grid=(EP // EB,),
        in_specs=[
            pl.BlockSpec((EB, 4), lambda i: (i, 0)),
            pl.BlockSpec((EB, DIM), lambda i: (i, 0)),
            pl.BlockSpec((4, 128), lambda i: (0, 0)),
            pl.BlockSpec((1, 128), lambda i: (0, 0)),
            pl.BlockSpec((128, DIM * DIM), lambda i: (0, 0)),
            pl.BlockSpec((1, DIM * DIM), lambda i: (0, 0)),
            pl.BlockSpec((DIM * DIM, DIM), lambda i: (0, 0)),
        ],
        out_specs=pl.BlockSpec((EB, DIM), lambda i: (i, 0)),
        out_shape=jax.ShapeDtypeStruct((EP, DIM), jnp.float32),
    )(ea_p, xj, We1T, be1.reshape(1, 128), We2pT, be2p, S)


def _update_body(p_ref, c_ref, x_ref, h_ref, wroot_ref, bconv_ref,
                 wih_ref, bih_ref, whh_ref, bhh_ref, o_ref):
    s = p_ref[0] + p_ref[1]
    cnt = c_ref[0][:, :1] + c_ref[1][:, :1]
    mean = s / jnp.maximum(cnt, 1.0)
    conv = (jnp.dot(x_ref[...], wroot_ref[...],
                    preferred_element_type=jnp.float32) + mean + bconv_ref[...])
    m = _leaky(conv)
    gi = jnp.dot(m, wih_ref[...], preferred_element_type=jnp.float32) + bih_ref[...]
    gh = jnp.dot(h_ref[...], whh_ref[...], preferred_element_type=jnp.float32) + bhh_ref[...]
    r = jax.nn.sigmoid(gi[:, :DIM] + gh[:, :DIM])
    z = jax.nn.sigmoid(gi[:, DIM:2 * DIM] + gh[:, DIM:2 * DIM])
    n = jnp.tanh(gi[:, 2 * DIM:] + r * gh[:, 2 * DIM:])
    hn = (1.0 - z) * n + z * h_ref[...]
    row = lax.broadcasted_iota(jnp.int32, (NP, 1), 0)
    o_ref[...] = jnp.where(row < N, hn, 0.0)


def _update(parts, cparts, x_cur, h_prev, WrootT, bconv, W_ihT, b_ih, W_hhT, b_hh):
    return pl.pallas_call(
        _update_body,
        out_shape=jax.ShapeDtypeStruct((NP, DIM), jnp.float32),
    )(parts, cparts, x_cur, h_prev, WrootT, bconv.reshape(1, DIM),
      W_ihT, b_ih.reshape(1, 3 * DIM), W_hhT, b_hh.reshape(1, 3 * DIM))


def _s2s_body(x_ref, b_ref, wlih_ref, blih_ref, wlhh_ref, blhh_ref,
              w1t_ref, b1_ref, o_ref, e_ref):
    q_star = jnp.zeros((B, 2 * DIM), jnp.float32)
    hx = jnp.zeros((B, DIM), jnp.float32)
    cx = jnp.zeros((B, DIM), jnp.float32)
    for _ in range(3):
        gates = (jnp.dot(q_star, wlih_ref[...], preferred_element_type=jnp.float32)
                 + blih_ref[...]
                 + jnp.dot(hx, wlhh_ref[...], preferred_element_type=jnp.float32)
                 + blhh_ref[...])
        i_g = jax.nn.sigmoid(gates[:, :DIM])
        f_g = jax.nn.sigmoid(gates[:, DIM:2 * DIM])
        g_g = jnp.tanh(gates[:, 2 * DIM:3 * DIM])
        o_g = jax.nn.sigmoid(gates[:, 3 * DIM:])
        cx = f_g * cx + i_g * g_g
        hx = o_g * jnp.tanh(cx)
        q = hx

        gid = lax.broadcasted_iota(jnp.int32, (CH, B), 1)
        emax = jnp.full((B,), -1e30, jnp.float32)
        for c in range(NC):
            x_c = x_ref[c * CH:(c + 1) * CH, :]
            b_c = b_ref[c * CH:(c + 1) * CH, :]
            mk = (b_c == gid)
            mf = mk.astype(jnp.float32)
            qg = jnp.dot(mf, q, preferred_element_type=jnp.float32)
            e_c = jnp.sum(x_c * qg, axis=1)
            em = jnp.where(mk, e_c[:, None], -1e30)
            emax = jnp.maximum(emax, jnp.max(em, axis=0))
            e_ref[c, :] = e_c
        denom = jnp.zeros((B,), jnp.float32)
        racc = jnp.zeros((B, DIM), jnp.float32)
        for c in range(NC):
            x_c = x_ref[c * CH:(c + 1) * CH, :]
            b_c = b_ref[c * CH:(c + 1) * CH, :]
            mf = (b_c == gid).astype(jnp.float32)
            me = jnp.sum(mf * emax[None, :], axis=1)
            ex = jnp.exp(e_ref[c, :] - me)
            denom = denom + jnp.dot(ex, mf, preferred_element_type=jnp.float32)
            racc = racc + lax.dot_general(mf, ex[:, None] * x_c,
                                          (((0,), (0,)), ((), ())),
                                          preferred_element_type=jnp.float32)
        r = racc / jnp.maximum(denom, 1e-30)[:, None]
        q_star = jnp.concatenate([q, r], axis=1)
    o_ref[...] = jnp.dot(q_star, w1t_ref[...],
                         preferred_element_type=jnp.float32) + b1_ref[...]


def _set2set(x_nodes, batch_p, Wl_ihT, bl_ih, Wl_hhT, bl_hh, W1T, b1):
    return pl.pallas_call(
        _s2s_body,
        out_shape=jax.ShapeDtypeStruct((B, 1), jnp.float32),
        scratch_shapes=[pltpu.VMEM((NC, CH), jnp.float32)],
    )(x_nodes, batch_p, Wl_ihT, bl_ih.reshape(1, 4 * DIM),
      Wl_hhT, bl_hh.reshape(1, 4 * DIM), W1T, b1.reshape(1, 1))


NW = 32
NJ = EP // NW // 128
RPT = NP // 16

_SC_MESH = plsc.VectorSubcoreMesh(core_axis_name="c", subcore_axis_name="s")
_SC_PARAMS = pltpu.CompilerParams(use_tc_tiling_on_sc=False)


def _sc_gather_body(x_hbm, idx_hbm, out_hbm, idx_v, rows_v, sem):
    wid = lax.axis_index("c") * 16 + lax.axis_index("s")
    pltpu.sync_copy(idx_hbm.at[wid], idx_v)

    @pl.loop(0, NJ)
    def _(j):
        pltpu.async_copy(x_hbm.at[idx_v.at[j]], rows_v, sem).wait()
        pltpu.sync_copy(rows_v, out_hbm.at[wid, j])


def _sc_gather(x_cur, idx_r):
    fn = pl.kernel(
        _sc_gather_body,
        out_type=jax.ShapeDtypeStruct((NW, NJ, 128, DIM), jnp.float32),
        mesh=_SC_MESH,
        compiler_params=_SC_PARAMS,
        scratch_types=[
            pltpu.VMEM((NJ, 128), jnp.int32),
            pltpu.VMEM((128, DIM), jnp.float32),
            pltpu.SemaphoreType.DMA,
        ],
    )
    return fn(x_cur, idx_r).reshape(EP, DIM)


def _sc_scatter_body(msg_hbm, idx_hbm, zero_hbm, out_hbm, idx_v, row_v, acc, sem):
    cid = lax.axis_index("c")
    sid = lax.axis_index("s")
    wid = cid * 16 + sid
    pltpu.sync_copy(zero_hbm.at[pl.ds(sid * RPT, RPT)],
                    acc.at[pl.ds(sid * RPT, RPT)])
    plsc.subcore_barrier()
    pltpu.sync_copy(idx_hbm.at[wid], idx_v)

    @pl.loop(0, NJ)
    def _(j):
        pltpu.sync_copy(msg_hbm.at[wid, j], row_v)
        pltpu.sync_copy(row_v, acc.at[idx_v.at[j]], add=True)

    plsc.subcore_barrier()
    pltpu.sync_copy(acc.at[pl.ds(sid * RPT, RPT)],
                    out_hbm.at[cid, pl.ds(sid * RPT, RPT)])


def _sc_scatter(msg, idx_r, zero):
    fn = pl.kernel(
        _sc_scatter_body,
        out_type=jax.ShapeDtypeStruct((2, NP, DIM), jnp.float32),
        mesh=_SC_MESH,
        compiler_params=_SC_PARAMS,
        scratch_types=[
            pltpu.VMEM((NJ, 128), jnp.int32),
            pltpu.VMEM((128, DIM), jnp.float32),
            pltpu.VMEM_SHARED((NP, DIM), jnp.float32),
            pltpu.SemaphoreType.DMA,
        ],
    )
    return fn(msg.reshape(NW, NJ, 128, DIM), idx_r, zero)


def kernel(x, edge_index, edge_attr, batch, W0, b0, We1, be1, We2, be2,
           Wroot, bconv, W_ih, W_hh, b_ih, b_hh, Wl_ih, Wl_hh, bl_ih, bl_hh,
           W1, b1):
    f32 = jnp.float32
    x_p = jnp.zeros((NP, NUM_FEAT), f32).at[:N].set(x)
    src_p = jnp.full((EP,), N, jnp.int32).at[:E].set(edge_index[0])
    dst_p = jnp.zeros((EP,), jnp.int32).at[:E].set(edge_index[1])
    ea_p = jnp.zeros((EP, 4), f32).at[:E].set(edge_attr)
    batch_p = jnp.full((NP, 1), B, jnp.int32).at[:N, 0].set(batch)

    We2p = We2.reshape(DIM, DIM, 128).transpose(1, 0, 2).reshape(DIM * DIM, 128)
    be2p = be2.reshape(DIM, DIM).T.reshape(1, DIM * DIM)
    S = jnp.repeat(jnp.eye(DIM, dtype=f32), DIM, axis=0)

    ones_msk = (jnp.arange(EP) < E).astype(f32)[:, None] * jnp.ones((1, DIM), f32)

    src_r = src_p.reshape(NW, NJ, 128)
    dst_r = dst_p.reshape(NW, NJ, 128)
    zero = jnp.zeros((NP, DIM), f32)

    out = _encode(x_p, W0.T, b0)

    cparts = _sc_scatter(ones_msk, dst_r, zero)

    h = out
    for _ in range(2):
        xj = _sc_gather(out, src_r)
        msg = _messages(ea_p, xj, We1.T, be1, We2p.T, be2p, S)
        parts = _sc_scatter(msg, dst_r, zero)
        h = _update(parts, cparts, out, h, Wroot, bconv,
                    W_ih.T, b_ih, W_hh.T, b_hh)
        out = h

    return _set2set(out, batch_p, Wl_ih.T, bl_ih, Wl_hh.T, bl_hh, W1.T, b1)


# speedup vs baseline: 3.5483x; 1.1073x over previous
import functools

import jax
import jax.numpy as jnp
from jax import lax
from jax.experimental import pallas as pl
from jax.experimental.pallas import tpu as pltpu
from jax.experimental.pallas import tpu_sc as plsc

N = 10000
E = 160000
NUM_FEAT = 128
DIM = 32
B = 312
NEG_SLOPE = 0.01

NP = 10240
EP = 163840
EB = 4096
CH = 1280
NC = NP // CH


def _leaky(v):
    return jnp.where(v >= 0, v, NEG_SLOPE * v)


def _encode_body(x_ref, w0t_ref, b0_ref, o_ref):
    v = _leaky(jnp.dot(x_ref[...], w0t_ref[...],
                       preferred_element_type=jnp.float32) + b0_ref[...])
    row = lax.broadcasted_iota(jnp.int32, (NP, 1), 0)
    o_ref[...] = jnp.where(row < N, v, 0.0)


def _encode(x_p, W0T, b0):
    return pl.pallas_call(
        _encode_body,
        out_shape=jax.ShapeDtypeStruct((NP, DIM), jnp.float32),
    )(x_p, W0T, b0.reshape(1, DIM))


def _msg_body(ea_ref, xj_ref, w1t_ref, b1_ref, w2t_ref, b2_ref, s_ref, o_ref):
    h = _leaky(jnp.dot(ea_ref[...], w1t_ref[...],
                       preferred_element_type=jnp.float32) + b1_ref[...])
    w = jnp.dot(h, w2t_ref[...], preferred_element_type=jnp.float32) + b2_ref[...]
    xt = jnp.tile(xj_ref[...], (1, DIM))
    o_ref[...] = jnp.dot(w * xt, s_ref[...], preferred_element_type=jnp.float32)


def _messages(ea_p, xj, We1T, be1, We2pT, be2p, S):
    return pl.pallas_call(
        _msg_body,
        grid=(EP // EB,),
        in_specs=[
            pl.BlockSpec((EB, 4), lambda i: (i, 0)),
            pl.BlockSpec((EB, DIM), lambda i: (i, 0)),
            pl.BlockSpec((4, 128), lambda i: (0, 0)),
            pl.BlockSpec((1, 128), lambda i: (0, 0)),
            pl.BlockSpec((128, DIM * DIM), lambda i: (0, 0)),
            pl.BlockSpec((1, DIM * DIM), lambda i: (0, 0)),
            pl.BlockSpec((DIM * DIM, DIM), lambda i: (0, 0)),
        ],
        out_specs=pl.BlockSpec((EB, DIM), lambda i: (i, 0)),
        out_shape=jax.ShapeDtypeStruct((EP, DIM), jnp.float32),
    )(ea_p, xj, We1T, be1.reshape(1, 128), We2pT, be2p, S)


def _update_body(p_ref, c_ref, x_ref, h_ref, wroot_ref, bconv_ref,
                 wih_ref, bih_ref, whh_ref, bhh_ref, o_ref):
    s = p_ref[0] + p_ref[1]
    cnt = c_ref[0][:, :1] + c_ref[1][:, :1]
    mean = s / jnp.maximum(cnt, 1.0)
    conv = (jnp.dot(x_ref[...], wroot_ref[...],
                    preferred_element_type=jnp.float32) + mean + bconv_ref[...])
    m = _leaky(conv)
    gi = jnp.dot(m, wih_ref[...], preferred_element_type=jnp.float32) + bih_ref[...]
    gh = jnp.dot(h_ref[...], whh_ref[...], preferred_element_type=jnp.float32) + bhh_ref[...]
    r = jax.nn.sigmoid(gi[:, :DIM] + gh[:, :DIM])
    z = jax.nn.sigmoid(gi[:, DIM:2 * DIM] + gh[:, DIM:2 * DIM])
    n = jnp.tanh(gi[:, 2 * DIM:] + r * gh[:, 2 * DIM:])
    hn = (1.0 - z) * n + z * h_ref[...]
    row = lax.broadcasted_iota(jnp.int32, (NP, 1), 0)
    o_ref[...] = jnp.where(row < N, hn, 0.0)


def _update(parts, cparts, x_cur, h_prev, WrootT, bconv, W_ihT, b_ih, W_hhT, b_hh):
    return pl.pallas_call(
        _update_body,
        out_shape=jax.ShapeDtypeStruct((NP, DIM), jnp.float32),
    )(parts, cparts, x_cur, h_prev, WrootT, bconv.reshape(1, DIM),
      W_ihT, b_ih.reshape(1, 3 * DIM), W_hhT, b_hh.reshape(1, 3 * DIM))


def _s2s_body(x_ref, b_ref, wlih_ref, blih_ref, wlhh_ref, blhh_ref,
              w1t_ref, b1_ref, o_ref, e_ref):
    q_star = jnp.zeros((B, 2 * DIM), jnp.float32)
    hx = jnp.zeros((B, DIM), jnp.float32)
    cx = jnp.zeros((B, DIM), jnp.float32)
    for _ in range(3):
        gates = (jnp.dot(q_star, wlih_ref[...], preferred_element_type=jnp.float32)
                 + blih_ref[...]
                 + jnp.dot(hx, wlhh_ref[...], preferred_element_type=jnp.float32)
                 + blhh_ref[...])
        i_g = jax.nn.sigmoid(gates[:, :DIM])
        f_g = jax.nn.sigmoid(gates[:, DIM:2 * DIM])
        g_g = jnp.tanh(gates[:, 2 * DIM:3 * DIM])
        o_g = jax.nn.sigmoid(gates[:, 3 * DIM:])
        cx = f_g * cx + i_g * g_g
        hx = o_g * jnp.tanh(cx)
        q = hx

        gid = lax.broadcasted_iota(jnp.int32, (CH, B), 1)
        emax = jnp.full((B,), -1e30, jnp.float32)
        for c in range(NC):
            x_c = x_ref[c * CH:(c + 1) * CH, :]
            b_c = b_ref[c * CH:(c + 1) * CH, :]
            mk = (b_c == gid)
            mf = mk.astype(jnp.float32)
            qg = jnp.dot(mf, q, preferred_element_type=jnp.float32)
            e_c = jnp.sum(x_c * qg, axis=1)
            em = jnp.where(mk, e_c[:, None], -1e30)
            emax = jnp.maximum(emax, jnp.max(em, axis=0))
            e_ref[c, :] = e_c
        denom = jnp.zeros((B,), jnp.float32)
        racc = jnp.zeros((B, DIM), jnp.float32)
        for c in range(NC):
            x_c = x_ref[c * CH:(c + 1) * CH, :]
            b_c = b_ref[c * CH:(c + 1) * CH, :]
            mf = (b_c == gid).astype(jnp.float32)
            me = jnp.sum(mf * emax[None, :], axis=1)
            ex = jnp.exp(e_ref[c, :] - me)
            denom = denom + jnp.dot(ex, mf, preferred_element_type=jnp.float32)
            racc = racc + lax.dot_general(mf, ex[:, None] * x_c,
                                          (((0,), (0,)), ((), ())),
                                          preferred_element_type=jnp.float32)
        r = racc / jnp.maximum(denom, 1e-30)[:, None]
        q_star = jnp.concatenate([q, r], axis=1)
    o_ref[...] = jnp.dot(q_star, w1t_ref[...],
                         preferred_element_type=jnp.float32) + b1_ref[...]


def _set2set(x_nodes, batch_p, Wl_ihT, bl_ih, Wl_hhT, bl_hh, W1T, b1):
    return pl.pallas_call(
        _s2s_body,
        out_shape=jax.ShapeDtypeStruct((B, 1), jnp.float32),
        scratch_shapes=[pltpu.VMEM((NC, CH), jnp.float32)],
    )(x_nodes, batch_p, Wl_ihT, bl_ih.reshape(1, 4 * DIM),
      Wl_hhT, bl_hh.reshape(1, 4 * DIM), W1T, b1.reshape(1, 1))


NW = 32
NJ = EP // NW // 128
RPT = NP // 16

_SC_MESH = plsc.VectorSubcoreMesh(core_axis_name="c", subcore_axis_name="s")
_SC_PARAMS = pltpu.CompilerParams(use_tc_tiling_on_sc=False)


def _sc_gather_body(x_hbm, idx_hbm, out_hbm, idx_v,
                    r0, r1, r2, r3, g0, g1, g2, g3, s0, s1, s2, s3):
    wid = lax.axis_index("c") * 16 + lax.axis_index("s")
    pltpu.sync_copy(idx_hbm.at[wid], idx_v)
    rows = (r0, r1, r2, r3)
    gsem = (g0, g1, g2, g3)
    ssem = (s0, s1, s2, s3)

    @pl.loop(0, NJ, step=4)
    def _(j):
        ds = [pltpu.async_copy(x_hbm.at[idx_v.at[j + k]], rows[k], gsem[k])
              for k in range(4)]
        ss = []
        for k in range(4):
            ds[k].wait()
            ss.append(pltpu.async_copy(rows[k], out_hbm.at[wid, j + k], ssem[k]))
        for k in range(4):
            ss[k].wait()


def _sc_gather(x_cur, idx_r):
    fn = pl.kernel(
        _sc_gather_body,
        out_type=jax.ShapeDtypeStruct((NW, NJ, 128, DIM), jnp.float32),
        mesh=_SC_MESH,
        compiler_params=_SC_PARAMS,
        scratch_types=[pltpu.VMEM((NJ, 128), jnp.int32)]
        + [pltpu.VMEM((128, DIM), jnp.float32)] * 4
        + [pltpu.SemaphoreType.DMA] * 8,
    )
    return fn(x_cur, idx_r).reshape(EP, DIM)


def _sc_scatter_body(msg_hbm, idx_hbm, zero_hbm, out_hbm,
                     idx_v, row0, row1, acc, sem0, sem1):
    cid = lax.axis_index("c")
    sid = lax.axis_index("s")
    wid = cid * 16 + sid
    pltpu.sync_copy(zero_hbm.at[pl.ds(sid * RPT, RPT)],
                    acc.at[pl.ds(sid * RPT, RPT)])
    plsc.subcore_barrier()
    pltpu.sync_copy(idx_hbm.at[wid], idx_v)

    @pl.loop(0, NJ, step=2)
    def _(j):
        l0 = pltpu.async_copy(msg_hbm.at[wid, j], row0, sem0)
        l1 = pltpu.async_copy(msg_hbm.at[wid, j + 1], row1, sem1)
        l0.wait()
        pltpu.sync_copy(row0, acc.at[idx_v.at[j]], add=True)
        l1.wait()
        pltpu.sync_copy(row1, acc.at[idx_v.at[j + 1]], add=True)

    plsc.subcore_barrier()
    pltpu.sync_copy(acc.at[pl.ds(sid * RPT, RPT)],
                    out_hbm.at[cid, pl.ds(sid * RPT, RPT)])


def _sc_scatter(msg, idx_r, zero):
    fn = pl.kernel(
        _sc_scatter_body,
        out_type=jax.ShapeDtypeStruct((2, NP, DIM), jnp.float32),
        mesh=_SC_MESH,
        compiler_params=_SC_PARAMS,
        scratch_types=[
            pltpu.VMEM((NJ, 128), jnp.int32),
            pltpu.VMEM((128, DIM), jnp.float32),
            pltpu.VMEM((128, DIM), jnp.float32),
            pltpu.VMEM_SHARED((NP, DIM), jnp.float32),
            pltpu.SemaphoreType.DMA,
            pltpu.SemaphoreType.DMA,
        ],
    )
    return fn(msg.reshape(NW, NJ, 128, DIM), idx_r, zero)


def _sc_count_body(idx_hbm, ones_hbm, zero_hbm, out_hbm, idx_v, ones_v, acc, sem):
    cid = lax.axis_index("c")
    sid = lax.axis_index("s")
    wid = cid * 16 + sid
    pltpu.sync_copy(zero_hbm.at[pl.ds(sid * RPT, RPT)],
                    acc.at[pl.ds(sid * RPT, RPT)])
    plsc.subcore_barrier()
    pltpu.sync_copy(idx_hbm.at[wid], idx_v)
    pltpu.sync_copy(ones_hbm, ones_v)

    @pl.loop(0, NJ)
    def _(j):
        pltpu.sync_copy(ones_v, acc.at[idx_v.at[j]], add=True)

    plsc.subcore_barrier()
    pltpu.sync_copy(acc.at[pl.ds(sid * RPT, RPT)],
                    out_hbm.at[cid, pl.ds(sid * RPT, RPT)])


def _sc_count(idx_r, ones_blk, zero):
    fn = pl.kernel(
        _sc_count_body,
        out_type=jax.ShapeDtypeStruct((2, NP, DIM), jnp.float32),
        mesh=_SC_MESH,
        compiler_params=_SC_PARAMS,
        scratch_types=[
            pltpu.VMEM((NJ, 128), jnp.int32),
            pltpu.VMEM((128, DIM), jnp.float32),
            pltpu.VMEM_SHARED((NP, DIM), jnp.float32),
            pltpu.SemaphoreType.DMA,
        ],
    )
    return fn(idx_r, ones_blk, zero)


def kernel(x, edge_index, edge_attr, batch, W0, b0, We1, be1, We2, be2,
           Wroot, bconv, W_ih, W_hh, b_ih, b_hh, Wl_ih, Wl_hh, bl_ih, bl_hh,
           W1, b1):
    f32 = jnp.float32
    x_p = jnp.zeros((NP, NUM_FEAT), f32).at[:N].set(x)
    src_p = jnp.full((EP,), N, jnp.int32).at[:E].set(edge_index[0])
    dst_p = jnp.full((EP,), NP - 1, jnp.int32).at[:E].set(edge_index[1])
    ea_p = jnp.zeros((EP, 4), f32).at[:E].set(edge_attr)
    batch_p = jnp.full((NP, 1), B, jnp.int32).at[:N, 0].set(batch)

    We2p = We2.reshape(DIM, DIM, 128).transpose(1, 0, 2).reshape(DIM * DIM, 128)
    be2p = be2.reshape(DIM, DIM).T.reshape(1, DIM * DIM)
    S = jnp.repeat(jnp.eye(DIM, dtype=f32), DIM, axis=0)

    ones_blk = jnp.ones((128, DIM), f32)

    src_r = src_p.reshape(NW, NJ, 128)
    dst_r = dst_p.reshape(NW, NJ, 128)
    zero = jnp.zeros((NP, DIM), f32)

    out = _encode(x_p, W0.T, b0)

    cparts = _sc_count(dst_r, ones_blk, zero)

    h = out
    for _ in range(2):
        xj = _sc_gather(out, src_r)
        msg = _messages(ea_p, xj, We1.T, be1, We2p.T, be2p, S)
        parts = _sc_scatter(msg, dst_r, zero)
        h = _update(parts, cparts, out, h, Wroot, bconv,
                    W_ih.T, b_ih, W_hh.T, b_hh)
        out = h

    return _set2set(out, batch_p, Wl_ih.T, bl_ih, Wl_hh.T, bl_hh, W1.T, b1)


# speedup vs baseline: 3.5566x; 1.0023x over previous
import functools

import jax
import jax.numpy as jnp
from jax import lax
from jax.experimental import pallas as pl
from jax.experimental.pallas import tpu as pltpu
from jax.experimental.pallas import tpu_sc as plsc

N = 10000
E = 160000
NUM_FEAT = 128
DIM = 32
B = 312
NEG_SLOPE = 0.01

NP = 10240
EP = 163840
EB = 4096
CH = 1280
NC = NP // CH


def _leaky(v):
    return jnp.where(v >= 0, v, NEG_SLOPE * v)


def _encode_body(x_ref, w0t_ref, b0_ref, o_ref):
    v = _leaky(jnp.dot(x_ref[...], w0t_ref[...],
                       preferred_element_type=jnp.float32) + b0_ref[...])
    row = lax.broadcasted_iota(jnp.int32, (NP, 1), 0)
    o_ref[...] = jnp.where(row < N, v, 0.0)


def _encode(x_p, W0T, b0):
    return pl.pallas_call(
        _encode_body,
        out_shape=jax.ShapeDtypeStruct((NP, DIM), jnp.float32),
    )(x_p, W0T, b0.reshape(1, DIM))


def _msg_body(ea_ref, xj_ref, w1t_ref, b1_ref, w2t_ref, b2_ref, s_ref, o_ref):
    h = _leaky(jnp.dot(ea_ref[...], w1t_ref[...],
                       preferred_element_type=jnp.float32) + b1_ref[...])
    w = jnp.dot(h.astype(jnp.bfloat16), w2t_ref[...],
                preferred_element_type=jnp.float32) + b2_ref[...]
    xt = jnp.tile(xj_ref[...], (1, DIM))
    o_ref[...] = jnp.dot(w * xt, s_ref[...], preferred_element_type=jnp.float32)


def _messages(ea_p, xj, We1T, be1, We2pT, be2p, S):
    return pl.pallas_call(
        _msg_body,
        grid=(EP // EB,),
        in_specs=[
            pl.BlockSpec((EB, 4), lambda i: (i, 0)),
            pl.BlockSpec((EB, DIM), lambda i: (i, 0)),
            pl.BlockSpec((4, 128), lambda i: (0, 0)),
            pl.BlockSpec((1, 128), lambda i: (0, 0)),
            pl.BlockSpec((128, DIM * DIM), lambda i: (0, 0)),
            pl.BlockSpec((1, DIM * DIM), lambda i: (0, 0)),
            pl.BlockSpec((DIM * DIM, DIM), lambda i: (0, 0)),
        ],
        out_specs=pl.BlockSpec((EB, DIM), lambda i: (i, 0)),
        out_shape=jax.ShapeDtypeStruct((EP, DIM), jnp.float32),
    )(ea_p, xj, We1T, be1.reshape(1, 128), We2pT, be2p, S)


def _update_body(p_ref, c_ref, x_ref, h_ref, wroot_ref, bconv_ref,
                 wih_ref, bih_ref, whh_ref, bhh_ref, o_ref):
    s = p_ref[0] + p_ref[1]
    cnt = c_ref[0][:, :1] + c_ref[1][:, :1]
    mean = s / jnp.maximum(cnt, 1.0)
    conv = (jnp.dot(x_ref[...], wroot_ref[...],
                    preferred_element_type=jnp.float32) + mean + bconv_ref[...])
    m = _leaky(conv)
    gi = jnp.dot(m, wih_ref[...], preferred_element_type=jnp.float32) + bih_ref[...]
    gh = jnp.dot(h_ref[...], whh_ref[...], preferred_element_type=jnp.float32) + bhh_ref[...]
    r = jax.nn.sigmoid(gi[:, :DIM] + gh[:, :DIM])
    z = jax.nn.sigmoid(gi[:, DIM:2 * DIM] + gh[:, DIM:2 * DIM])
    n = jnp.tanh(gi[:, 2 * DIM:] + r * gh[:, 2 * DIM:])
    hn = (1.0 - z) * n + z * h_ref[...]
    row = lax.broadcasted_iota(jnp.int32, (NP, 1), 0)
    o_ref[...] = jnp.where(row < N, hn, 0.0)


def _update(parts, cparts, x_cur, h_prev, WrootT, bconv, W_ihT, b_ih, W_hhT, b_hh):
    return pl.pallas_call(
        _update_body,
        out_shape=jax.ShapeDtypeStruct((NP, DIM), jnp.float32),
    )(parts, cparts, x_cur, h_prev, WrootT, bconv.reshape(1, DIM),
      W_ihT, b_ih.reshape(1, 3 * DIM), W_hhT, b_hh.reshape(1, 3 * DIM))


def _s2s_body(x_ref, b_ref, wlih_ref, blih_ref, wlhh_ref, blhh_ref,
              w1t_ref, b1_ref, o_ref, e_ref):
    q_star = jnp.zeros((B, 2 * DIM), jnp.float32)
    hx = jnp.zeros((B, DIM), jnp.float32)
    cx = jnp.zeros((B, DIM), jnp.float32)
    for _ in range(3):
        gates = (jnp.dot(q_star, wlih_ref[...], preferred_element_type=jnp.float32)
                 + blih_ref[...]
                 + jnp.dot(hx, wlhh_ref[...], preferred_element_type=jnp.float32)
                 + blhh_ref[...])
        i_g = jax.nn.sigmoid(gates[:, :DIM])
        f_g = jax.nn.sigmoid(gates[:, DIM:2 * DIM])
        g_g = jnp.tanh(gates[:, 2 * DIM:3 * DIM])
        o_g = jax.nn.sigmoid(gates[:, 3 * DIM:])
        cx = f_g * cx + i_g * g_g
        hx = o_g * jnp.tanh(cx)
        q = hx

        gid = lax.broadcasted_iota(jnp.int32, (CH, B), 1)
        emax = jnp.full((B,), -1e30, jnp.float32)
        for c in range(NC):
            x_c = x_ref[c * CH:(c + 1) * CH, :]
            b_c = b_ref[c * CH:(c + 1) * CH, :]
            mk = (b_c == gid)
            mf = mk.astype(jnp.float32)
            qg = jnp.dot(mf, q, preferred_element_type=jnp.float32)
            e_c = jnp.sum(x_c * qg, axis=1)
            em = jnp.where(mk, e_c[:, None], -1e30)
            emax = jnp.maximum(emax, jnp.max(em, axis=0))
            e_ref[c, :] = e_c
        denom = jnp.zeros((B,), jnp.float32)
        racc = jnp.zeros((B, DIM), jnp.float32)
        for c in range(NC):
            x_c = x_ref[c * CH:(c + 1) * CH, :]
            b_c = b_ref[c * CH:(c + 1) * CH, :]
            mf = (b_c == gid).astype(jnp.float32)
            me = jnp.sum(mf * emax[None, :], axis=1)
            ex = jnp.exp(e_ref[c, :] - me)
            denom = denom + jnp.dot(ex, mf, preferred_element_type=jnp.float32)
            racc = racc + lax.dot_general(mf, ex[:, None] * x_c,
                                          (((0,), (0,)), ((), ())),
                                          preferred_element_type=jnp.float32)
        r = racc / jnp.maximum(denom, 1e-30)[:, None]
        q_star = jnp.concatenate([q, r], axis=1)
    o_ref[...] = jnp.dot(q_star, w1t_ref[...],
                         preferred_element_type=jnp.float32) + b1_ref[...]


def _set2set(x_nodes, batch_p, Wl_ihT, bl_ih, Wl_hhT, bl_hh, W1T, b1):
    return pl.pallas_call(
        _s2s_body,
        out_shape=jax.ShapeDtypeStruct((B, 1), jnp.float32),
        scratch_shapes=[pltpu.VMEM((NC, CH), jnp.float32)],
    )(x_nodes, batch_p, Wl_ihT, bl_ih.reshape(1, 4 * DIM),
      Wl_hhT, bl_hh.reshape(1, 4 * DIM), W1T, b1.reshape(1, 1))


NW = 32
NJ = EP // NW // 128
RPT = NP // 16

_SC_MESH = plsc.VectorSubcoreMesh(core_axis_name="c", subcore_axis_name="s")
_SC_PARAMS = pltpu.CompilerParams(use_tc_tiling_on_sc=False)


def _sc_gather_body(x_hbm, idx_hbm, out_hbm, idx_v,
                    r0, r1, r2, r3, g0, g1, g2, g3, s0, s1, s2, s3):
    wid = lax.axis_index("c") * 16 + lax.axis_index("s")
    pltpu.sync_copy(idx_hbm.at[wid], idx_v)
    rows = (r0, r1, r2, r3)
    gsem = (g0, g1, g2, g3)
    ssem = (s0, s1, s2, s3)

    @pl.loop(0, NJ, step=4)
    def _(j):
        ds = [pltpu.async_copy(x_hbm.at[idx_v.at[j + k]], rows[k], gsem[k])
              for k in range(4)]
        ss = []
        for k in range(4):
            ds[k].wait()
            ss.append(pltpu.async_copy(rows[k], out_hbm.at[wid, j + k], ssem[k]))
        for k in range(4):
            ss[k].wait()


def _sc_gather(x_cur, idx_r):
    fn = pl.kernel(
        _sc_gather_body,
        out_type=jax.ShapeDtypeStruct((NW, NJ, 128, DIM), jnp.float32),
        mesh=_SC_MESH,
        compiler_params=_SC_PARAMS,
        scratch_types=[pltpu.VMEM((NJ, 128), jnp.int32)]
        + [pltpu.VMEM((128, DIM), jnp.float32)] * 4
        + [pltpu.SemaphoreType.DMA] * 8,
    )
    return fn(x_cur, idx_r).reshape(EP, DIM)


def _sc_scatter_body(msg_hbm, idx_hbm, zero_hbm, out_hbm,
                     idx_v, row0, row1, acc, sem0, sem1):
    cid = lax.axis_index("c")
    sid = lax.axis_index("s")
    wid = cid * 16 + sid
    pltpu.sync_copy(zero_hbm.at[pl.ds(sid * RPT, RPT)],
                    acc.at[pl.ds(sid * RPT, RPT)])
    plsc.subcore_barrier()
    pltpu.sync_copy(idx_hbm.at[wid], idx_v)

    @pl.loop(0, NJ, step=2)
    def _(j):
        l0 = pltpu.async_copy(msg_hbm.at[wid, j], row0, sem0)
        l1 = pltpu.async_copy(msg_hbm.at[wid, j + 1], row1, sem1)
        l0.wait()
        pltpu.sync_copy(row0, acc.at[idx_v.at[j]], add=True)
        l1.wait()
        pltpu.sync_copy(row1, acc.at[idx_v.at[j + 1]], add=True)

    plsc.subcore_barrier()
    pltpu.sync_copy(acc.at[pl.ds(sid * RPT, RPT)],
                    out_hbm.at[cid, pl.ds(sid * RPT, RPT)])


def _sc_scatter(msg, idx_r, zero):
    fn = pl.kernel(
        _sc_scatter_body,
        out_type=jax.ShapeDtypeStruct((2, NP, DIM), jnp.float32),
        mesh=_SC_MESH,
        compiler_params=_SC_PARAMS,
        scratch_types=[
            pltpu.VMEM((NJ, 128), jnp.int32),
            pltpu.VMEM((128, DIM), jnp.float32),
            pltpu.VMEM((128, DIM), jnp.float32),
            pltpu.VMEM_SHARED((NP, DIM), jnp.float32),
            pltpu.SemaphoreType.DMA,
            pltpu.SemaphoreType.DMA,
        ],
    )
    return fn(msg.reshape(NW, NJ, 128, DIM), idx_r, zero)


def _sc_count_body(idx_hbm, ones_hbm, zero_hbm, out_hbm, idx_v, ones_v, acc, sem):
    cid = lax.axis_index("c")
    sid = lax.axis_index("s")
    wid = cid * 16 + sid
    pltpu.sync_copy(zero_hbm.at[pl.ds(sid * RPT, RPT)],
                    acc.at[pl.ds(sid * RPT, RPT)])
    plsc.subcore_barrier()
    pltpu.sync_copy(idx_hbm.at[wid], idx_v)
    pltpu.sync_copy(ones_hbm, ones_v)

    @pl.loop(0, NJ)
    def _(j):
        pltpu.sync_copy(ones_v, acc.at[idx_v.at[j]], add=True)

    plsc.subcore_barrier()
    pltpu.sync_copy(acc.at[pl.ds(sid * RPT, RPT)],
                    out_hbm.at[cid, pl.ds(sid * RPT, RPT)])


def _sc_count(idx_r, ones_blk, zero):
    fn = pl.kernel(
        _sc_count_body,
        out_type=jax.ShapeDtypeStruct((2, NP, DIM), jnp.float32),
        mesh=_SC_MESH,
        compiler_params=_SC_PARAMS,
        scratch_types=[
            pltpu.VMEM((NJ, 128), jnp.int32),
            pltpu.VMEM((128, DIM), jnp.float32),
            pltpu.VMEM_SHARED((NP, DIM), jnp.float32),
            pltpu.SemaphoreType.DMA,
        ],
    )
    return fn(idx_r, ones_blk, zero)


def kernel(x, edge_index, edge_attr, batch, W0, b0, We1, be1, We2, be2,
           Wroot, bconv, W_ih, W_hh, b_ih, b_hh, Wl_ih, Wl_hh, bl_ih, bl_hh,
           W1, b1):
    f32 = jnp.float32
    x_p = jnp.zeros((NP, NUM_FEAT), f32).at[:N].set(x)
    src_p = jnp.full((EP,), N, jnp.int32).at[:E].set(edge_index[0])
    dst_p = jnp.full((EP,), NP - 1, jnp.int32).at[:E].set(edge_index[1])
    ea_p = jnp.zeros((EP, 4), f32).at[:E].set(edge_attr)
    batch_p = jnp.full((NP, 1), B, jnp.int32).at[:N, 0].set(batch)

    We2p = We2.reshape(DIM, DIM, 128).transpose(1, 0, 2).reshape(DIM * DIM, 128)
    be2p = be2.reshape(DIM, DIM).T.reshape(1, DIM * DIM)
    S = jnp.repeat(jnp.eye(DIM, dtype=f32), DIM, axis=0)

    ones_blk = jnp.ones((128, DIM), f32)

    src_r = src_p.reshape(NW, NJ, 128)
    dst_r = dst_p.reshape(NW, NJ, 128)
    zero = jnp.zeros((NP, DIM), f32)

    out = _encode(x_p, W0.T, b0)
    We2pT_bf = We2p.T.astype(jnp.bfloat16)

    cparts = _sc_count(dst_r, ones_blk, zero)

    h = out
    for _ in range(2):
        xj = _sc_gather(out, src_r)
        msg = _messages(ea_p, xj, We1.T, be1, We2pT_bf, be2p, S)
        parts = _sc_scatter(msg, dst_r, zero)
        h = _update(parts, cparts, out, h, Wroot, bconv,
                    W_ih.T, b_ih, W_hh.T, b_hh)
        out = h

    return _set2set(out, batch_p, Wl_ih.T, bl_ih, Wl_hh.T, bl_hh, W1.T, b1)


# speedup vs baseline: 3.5596x; 1.0008x over previous
import jax
import jax.numpy as jnp
from jax import lax
from jax.experimental import pallas as pl
from jax.experimental.pallas import tpu as pltpu
from jax.experimental.pallas import tpu_sc as plsc

N = 10000
E = 160000
NUM_FEAT = 128
DIM = 32
B = 312
NEG_SLOPE = 0.01

NP = 10240
EP = 163840
EB = 4096
CH = 1280
NC = NP // CH


def _leaky(v):
    return jnp.where(v >= 0, v, NEG_SLOPE * v)


def _encode_body(x_ref, w0t_ref, b0_ref, o_ref):
    v = _leaky(jnp.dot(x_ref[...], w0t_ref[...],
                       preferred_element_type=jnp.float32) + b0_ref[...])
    row = lax.broadcasted_iota(jnp.int32, (NP, 1), 0)
    o_ref[...] = jnp.where(row < N, v, 0.0)


def _encode(x_p, W0T, b0):
    return pl.pallas_call(
        _encode_body,
        out_shape=jax.ShapeDtypeStruct((NP, DIM), jnp.float32),
    )(x_p, W0T, b0.reshape(1, DIM))


def _msg_body(ea_ref, xj_ref, w1t_ref, b1_ref, w2t_ref, b2_ref, s_ref, o_ref):
    h = _leaky(jnp.dot(ea_ref[...], w1t_ref[...],
                       preferred_element_type=jnp.float32) + b1_ref[...])
    w = jnp.dot(h.astype(jnp.bfloat16), w2t_ref[...],
                preferred_element_type=jnp.float32) + b2_ref[...]
    xt = jnp.tile(xj_ref[...], (1, DIM))
    o_ref[...] = jnp.dot(w * xt, s_ref[...], preferred_element_type=jnp.float32)


def _messages(ea_p, xj, We1T, be1, We2pT, be2p, S):
    return pl.pallas_call(
        _msg_body,
        grid=(EP // EB,),
        in_specs=[
            pl.BlockSpec((EB, 4), lambda i: (i, 0)),
            pl.BlockSpec((EB, DIM), lambda i: (i, 0)),
            pl.BlockSpec((4, 128), lambda i: (0, 0)),
            pl.BlockSpec((1, 128), lambda i: (0, 0)),
            pl.BlockSpec((128, DIM * DIM), lambda i: (0, 0)),
            pl.BlockSpec((1, DIM * DIM), lambda i: (0, 0)),
            pl.BlockSpec((DIM * DIM, DIM), lambda i: (0, 0)),
        ],
        out_specs=pl.BlockSpec((EB, DIM), lambda i: (i, 0)),
        out_shape=jax.ShapeDtypeStruct((EP, DIM), jnp.float32),
    )(ea_p, xj, We1T, be1.reshape(1, 128), We2pT, be2p, S)


def _update_body(p_ref, c_ref, x_ref, h_ref, wroot_ref, bconv_ref,
                 wih_ref, bih_ref, whh_ref, bhh_ref, o_ref):
    s = p_ref[0] + p_ref[1]
    cnt = c_ref[0][:, :1] + c_ref[1][:, :1]
    mean = s / jnp.maximum(cnt, 1.0)
    conv = (jnp.dot(x_ref[...], wroot_ref[...],
                    preferred_element_type=jnp.float32) + mean + bconv_ref[...])
    m = _leaky(conv)
    gi = jnp.dot(m, wih_ref[...], preferred_element_type=jnp.float32) + bih_ref[...]
    gh = jnp.dot(h_ref[...], whh_ref[...], preferred_element_type=jnp.float32) + bhh_ref[...]
    r = jax.nn.sigmoid(gi[:, :DIM] + gh[:, :DIM])
    z = jax.nn.sigmoid(gi[:, DIM:2 * DIM] + gh[:, DIM:2 * DIM])
    n = jnp.tanh(gi[:, 2 * DIM:] + r * gh[:, 2 * DIM:])
    hn = (1.0 - z) * n + z * h_ref[...]
    row = lax.broadcasted_iota(jnp.int32, (NP, 1), 0)
    o_ref[...] = jnp.where(row < N, hn, 0.0)


def _update(parts, cparts, x_cur, h_prev, WrootT, bconv, W_ihT, b_ih, W_hhT, b_hh):
    return pl.pallas_call(
        _update_body,
        out_shape=jax.ShapeDtypeStruct((NP, DIM), jnp.float32),
    )(parts, cparts, x_cur, h_prev, WrootT, bconv.reshape(1, DIM),
      W_ihT, b_ih.reshape(1, 3 * DIM), W_hhT, b_hh.reshape(1, 3 * DIM))


def _s2s_body(x_ref, b_ref, wlih_ref, blih_ref, wlhh_ref, blhh_ref,
              w1t_ref, b1_ref, o_ref, e_ref):
    q_star = jnp.zeros((B, 2 * DIM), jnp.float32)
    hx = jnp.zeros((B, DIM), jnp.float32)
    cx = jnp.zeros((B, DIM), jnp.float32)
    for _ in range(3):
        gates = (jnp.dot(q_star, wlih_ref[...], preferred_element_type=jnp.float32)
                 + blih_ref[...]
                 + jnp.dot(hx, wlhh_ref[...], preferred_element_type=jnp.float32)
                 + blhh_ref[...])
        i_g = jax.nn.sigmoid(gates[:, :DIM])
        f_g = jax.nn.sigmoid(gates[:, DIM:2 * DIM])
        g_g = jnp.tanh(gates[:, 2 * DIM:3 * DIM])
        o_g = jax.nn.sigmoid(gates[:, 3 * DIM:])
        cx = f_g * cx + i_g * g_g
        hx = o_g * jnp.tanh(cx)
        q = hx

        gid = lax.broadcasted_iota(jnp.int32, (CH, B), 1)
        emax = jnp.full((B,), -1e30, jnp.float32)
        for c in range(NC):
            x_c = x_ref[c * CH:(c + 1) * CH, :]
            b_c = b_ref[c * CH:(c + 1) * CH, :]
            mk = (b_c == gid)
            mf = mk.astype(jnp.float32)
            qg = jnp.dot(mf, q, preferred_element_type=jnp.float32)
            e_c = jnp.sum(x_c * qg, axis=1)
            em = jnp.where(mk, e_c[:, None], -1e30)
            emax = jnp.maximum(emax, jnp.max(em, axis=0))
            e_ref[c, :] = e_c
        denom = jnp.zeros((B,), jnp.float32)
        racc = jnp.zeros((B, DIM), jnp.float32)
        for c in range(NC):
            x_c = x_ref[c * CH:(c + 1) * CH, :]
            b_c = b_ref[c * CH:(c + 1) * CH, :]
            mf = (b_c == gid).astype(jnp.float32)
            me = jnp.sum(mf * emax[None, :], axis=1)
            ex = jnp.exp(e_ref[c, :] - me)
            denom = denom + jnp.dot(ex, mf, preferred_element_type=jnp.float32)
            racc = racc + lax.dot_general(mf, ex[:, None] * x_c,
                                          (((0,), (0,)), ((), ())),
                                          preferred_element_type=jnp.float32)
        r = racc / jnp.maximum(denom, 1e-30)[:, None]
        q_star = jnp.concatenate([q, r], axis=1)
    o_ref[...] = jnp.dot(q_star, w1t_ref[...],
                         preferred_element_type=jnp.float32) + b1_ref[...]


def _set2set(x_nodes, batch_p, Wl_ihT, bl_ih, Wl_hhT, bl_hh, W1T, b1):
    return pl.pallas_call(
        _s2s_body,
        out_shape=jax.ShapeDtypeStruct((B, 1), jnp.float32),
        scratch_shapes=[pltpu.VMEM((NC, CH), jnp.float32)],
    )(x_nodes, batch_p, Wl_ihT, bl_ih.reshape(1, 4 * DIM),
      Wl_hhT, bl_hh.reshape(1, 4 * DIM), W1T, b1.reshape(1, 1))


NW = 32
NJ = EP // NW // 128
RPT = NP // 16

_SC_MESH = plsc.VectorSubcoreMesh(core_axis_name="c", subcore_axis_name="s")
_SC_PARAMS = pltpu.CompilerParams(use_tc_tiling_on_sc=False)


def _sc_gather_body(x_hbm, idx_hbm, out_hbm, idx_v,
                    r0, r1, r2, r3, g0, g1, g2, g3, s0, s1, s2, s3):
    wid = lax.axis_index("c") * 16 + lax.axis_index("s")
    pltpu.sync_copy(idx_hbm.at[wid], idx_v)
    rows = (r0, r1, r2, r3)
    gsem = (g0, g1, g2, g3)
    ssem = (s0, s1, s2, s3)

    @pl.loop(0, NJ, step=4)
    def _(j):
        ds = [pltpu.async_copy(x_hbm.at[idx_v.at[j + k]], rows[k], gsem[k])
              for k in range(4)]
        ss = []
        for k in range(4):
            ds[k].wait()
            ss.append(pltpu.async_copy(rows[k], out_hbm.at[wid, j + k], ssem[k]))
        for k in range(4):
            ss[k].wait()


def _sc_gather(x_cur, idx_r):
    fn = pl.kernel(
        _sc_gather_body,
        out_type=jax.ShapeDtypeStruct((NW, NJ, 128, DIM), jnp.float32),
        mesh=_SC_MESH,
        compiler_params=_SC_PARAMS,
        scratch_types=[pltpu.VMEM((NJ, 128), jnp.int32)]
        + [pltpu.VMEM((128, DIM), jnp.float32)] * 4
        + [pltpu.SemaphoreType.DMA] * 8,
    )
    return fn(x_cur, idx_r).reshape(EP, DIM)


def _sc_scatter_body(msg_hbm, idx_hbm, zero_hbm, out_hbm,
                     idx_v, row0, row1, acc, sem0, sem1):
    cid = lax.axis_index("c")
    sid = lax.axis_index("s")
    wid = cid * 16 + sid
    pltpu.sync_copy(zero_hbm.at[pl.ds(sid * RPT, RPT)],
                    acc.at[pl.ds(sid * RPT, RPT)])
    plsc.subcore_barrier()
    pltpu.sync_copy(idx_hbm.at[wid], idx_v)

    @pl.loop(0, NJ, step=2)
    def _(j):
        l0 = pltpu.async_copy(msg_hbm.at[wid, j], row0, sem0)
        l1 = pltpu.async_copy(msg_hbm.at[wid, j + 1], row1, sem1)
        l0.wait()
        pltpu.sync_copy(row0, acc.at[idx_v.at[j]], add=True)
        l1.wait()
        pltpu.sync_copy(row1, acc.at[idx_v.at[j + 1]], add=True)

    plsc.subcore_barrier()
    pltpu.sync_copy(acc.at[pl.ds(sid * RPT, RPT)],
                    out_hbm.at[cid, pl.ds(sid * RPT, RPT)])


def _sc_scatter(msg, idx_r, zero):
    fn = pl.kernel(
        _sc_scatter_body,
        out_type=jax.ShapeDtypeStruct((2, NP, DIM), jnp.float32),
        mesh=_SC_MESH,
        compiler_params=_SC_PARAMS,
        scratch_types=[
            pltpu.VMEM((NJ, 128), jnp.int32),
            pltpu.VMEM((128, DIM), jnp.float32),
            pltpu.VMEM((128, DIM), jnp.float32),
            pltpu.VMEM_SHARED((NP, DIM), jnp.float32),
            pltpu.SemaphoreType.DMA,
            pltpu.SemaphoreType.DMA,
        ],
    )
    return fn(msg.reshape(NW, NJ, 128, DIM), idx_r, zero)


def _sc_count_body(idx_hbm, ones_hbm, zero_hbm, out_hbm, idx_v, ones_v, acc, sem):
    cid = lax.axis_index("c")
    sid = lax.axis_index("s")
    wid = cid * 16 + sid
    pltpu.sync_copy(zero_hbm.at[pl.ds(sid * RPT, RPT)],
                    acc.at[pl.ds(sid * RPT, RPT)])
    plsc.subcore_barrier()
    pltpu.sync_copy(idx_hbm.at[wid], idx_v)
    pltpu.sync_copy(ones_hbm, ones_v)

    @pl.loop(0, NJ)
    def _(j):
        pltpu.sync_copy(ones_v, acc.at[idx_v.at[j]], add=True)

    plsc.subcore_barrier()
    pltpu.sync_copy(acc.at[pl.ds(sid * RPT, RPT)],
                    out_hbm.at[cid, pl.ds(sid * RPT, RPT)])


def _sc_count(idx_r, ones_blk, zero):
    fn = pl.kernel(
        _sc_count_body,
        out_type=jax.ShapeDtypeStruct((2, NP, DIM), jnp.float32),
        mesh=_SC_MESH,
        compiler_params=_SC_PARAMS,
        scratch_types=[
            pltpu.VMEM((NJ, 128), jnp.int32),
            pltpu.VMEM((128, DIM), jnp.float32),
            pltpu.VMEM_SHARED((NP, DIM), jnp.float32),
            pltpu.SemaphoreType.DMA,
        ],
    )
    return fn(idx_r, ones_blk, zero)


def kernel(x, edge_index, edge_attr, batch, W0, b0, We1, be1, We2, be2,
           Wroot, bconv, W_ih, W_hh, b_ih, b_hh, Wl_ih, Wl_hh, bl_ih, bl_hh,
           W1, b1):
    f32 = jnp.float32
    x_p = jnp.zeros((NP, NUM_FEAT), f32).at[:N].set(x)
    src_p = jnp.full((EP,), N, jnp.int32).at[:E].set(edge_index[0])
    dst_p = jnp.full((EP,), NP - 1, jnp.int32).at[:E].set(edge_index[1])
    ea_p = jnp.zeros((EP, 4), f32).at[:E].set(edge_attr)
    batch_p = jnp.full((NP, 1), B, jnp.int32).at[:N, 0].set(batch)

    We2p = We2.reshape(DIM, DIM, 128).transpose(1, 0, 2).reshape(DIM * DIM, 128)
    be2p = be2.reshape(DIM, DIM).T.reshape(1, DIM * DIM)
    S = jnp.repeat(jnp.eye(DIM, dtype=f32), DIM, axis=0)

    ones_blk = jnp.ones((128, DIM), f32)

    src_r = src_p.reshape(NW, NJ, 128)
    dst_r = dst_p.reshape(NW, NJ, 128)
    zero = jnp.zeros((NP, DIM), f32)

    out = _encode(x_p, W0.T, b0)
    We2pT_bf = We2p.T.astype(jnp.bfloat16)

    cparts = _sc_count(dst_r, ones_blk, zero)

    h = out
    for _ in range(2):
        xj = _sc_gather(out, src_r)
        msg = _messages(ea_p, xj, We1.T, be1, We2pT_bf, be2p, S)
        parts = _sc_scatter(msg, dst_r, zero)
        h = _update(parts, cparts, out, h, Wroot, bconv,
                    W_ih.T, b_ih, W_hh.T, b_hh)
        out = h

    return _set2set(out, batch_p, Wl_ih.T, bl_ih, Wl_hh.T, bl_hh, W1.T, b1)
